# Initial kernel scaffold; baseline (speedup 1.0000x reference)
import functools

import jax
import jax.numpy as jnp
from jax import lax
from jax.experimental import pallas as pl
from jax.experimental.pallas import tpu as pltpu
from jax.experimental.pallas import tpu_sc as plsc

_N = 10000
_E = 320000
_D = 128
_H = 128
_OUT = 10
_G = 32

_NC = 2
_NS = 16
_LANES = 16
_NW = _NC * _NS

_E_TILE = _E // _NW
_CHUNK = 80
_RPS = _N // _NS
_ZROWS = 125

_mesh = plsc.VectorSubcoreMesh(core_axis_name="c", subcore_axis_name="s")


@functools.partial(
    pl.kernel,
    out_type=jax.ShapeDtypeStruct((_NC, _N, _LANES), jnp.float32),
    mesh=_mesh,
    scratch_types=[
        pltpu.VMEM((_CHUNK,), jnp.int32),
        pltpu.VMEM((_CHUNK, _LANES), jnp.float32),
        pltpu.VMEM((_ZROWS, _LANES), jnp.float32),
        pltpu.VMEM_SHARED((_N, _LANES), jnp.float32),
    ],
)
def _deg_kernel(col_hbm, deg_hbm, idx_v, ones_v, zeros_v, deg_sh):
    cid = lax.axis_index("c")
    sid = lax.axis_index("s")

    @pl.loop(0, _CHUNK)
    def _(i):
        ones_v[i, :] = jnp.full((_LANES,), 1.0, jnp.float32)

    @pl.loop(0, _ZROWS)
    def _(i):
        zeros_v[i, :] = jnp.zeros((_LANES,), jnp.float32)

    @pl.loop(0, _RPS, step=_ZROWS)
    def _(r):
        pltpu.sync_copy(zeros_v, deg_sh.at[pl.ds(sid * _RPS + r, _ZROWS)])

    plsc.subcore_barrier()

    base = (cid * _NS + sid) * _E_TILE

    @pl.loop(0, _E_TILE, step=_CHUNK)
    def _(off):
        pltpu.sync_copy(col_hbm.at[pl.ds(base + off, _CHUNK)], idx_v)
        pltpu.sync_copy(ones_v, deg_sh.at[idx_v], add=True)

    plsc.subcore_barrier()
    pltpu.sync_copy(
        deg_sh.at[pl.ds(sid * _RPS, _RPS)],
        deg_hbm.at[cid, pl.ds(sid * _RPS, _RPS)],
    )


@functools.partial(
    pl.kernel,
    out_type=jax.ShapeDtypeStruct((_NC, _N, _H), jnp.float32),
    mesh=_mesh,
    scratch_types=[
        pltpu.VMEM((_CHUNK,), jnp.int32),
        pltpu.VMEM((_CHUNK,), jnp.int32),
        pltpu.VMEM((_CHUNK, _H), jnp.float32),
        pltpu.VMEM((_ZROWS, _H), jnp.float32),
        pltpu.VMEM_SHARED((_N, _H), jnp.float32),
        pltpu.SemaphoreType.DMA,
    ],
)
def _gs_kernel(row_hbm, col_hbm, y_hbm, acc_hbm,
               ridx_v, cidx_v, rows_v, zeros_v, acc_sh, sem):
    cid = lax.axis_index("c")
    sid = lax.axis_index("s")

    @pl.loop(0, _ZROWS)
    def _(i):
        @pl.loop(0, _H, step=_LANES)
        def _(j):
            zeros_v[i, pl.ds(j, _LANES)] = jnp.zeros((_LANES,), jnp.float32)

    @pl.loop(0, _RPS, step=_ZROWS)
    def _(r):
        pltpu.sync_copy(zeros_v, acc_sh.at[pl.ds(sid * _RPS + r, _ZROWS)])

    plsc.subcore_barrier()

    base = (cid * _NS + sid) * _E_TILE

    @pl.loop(0, _E_TILE, step=_CHUNK)
    def _(off):
        pltpu.sync_copy(row_hbm.at[pl.ds(base + off, _CHUNK)], ridx_v)
        pltpu.sync_copy(col_hbm.at[pl.ds(base + off, _CHUNK)], cidx_v)
        pltpu.async_copy(y_hbm.at[ridx_v], rows_v, sem).wait()
        pltpu.sync_copy(rows_v, acc_sh.at[cidx_v], add=True)

    plsc.subcore_barrier()
    pltpu.sync_copy(
        acc_sh.at[pl.ds(sid * _RPS, _RPS)],
        acc_hbm.at[cid, pl.ds(sid * _RPS, _RPS)],
    )


_RB = 1000


def _xw_body(x_ref, w_ref, o_ref):
    o_ref[...] = jnp.dot(x_ref[...], w_ref[...],
                         preferred_element_type=jnp.float32)


_xw_call = pl.pallas_call(
    _xw_body,
    grid=(_N // _RB,),
    in_specs=[
        pl.BlockSpec((_RB, _D), lambda i: (i, 0)),
        pl.BlockSpec((_D, _H), lambda i: (0, 0)),
    ],
    out_specs=pl.BlockSpec((_RB, _H), lambda i: (i, 0)),
    out_shape=jax.ShapeDtypeStruct((_N, _H), jnp.float32),
)


def _scale_body(dp_ref, xw_ref, y_ref):
    deg = dp_ref[0, :, 0:1] + dp_ref[1, :, 0:1] + 1.0
    y_ref[...] = lax.rsqrt(deg) * xw_ref[...]


_scale_call = pl.pallas_call(
    _scale_body,
    grid=(_N // _RB,),
    in_specs=[
        pl.BlockSpec((_NC, _RB, _LANES), lambda i: (0, i, 0)),
        pl.BlockSpec((_RB, _H), lambda i: (i, 0)),
    ],
    out_specs=pl.BlockSpec((_RB, _H), lambda i: (i, 0)),
    out_shape=jax.ShapeDtypeStruct((_N, _H), jnp.float32),
)


def _epi_body(dp_ref, acc_ref, y_ref, b_ref, b1_ref, w2_ref, b2_ref,
              logits_ref, pool_ref, pool_acc):
    i = pl.program_id(0)

    @pl.when(i == 0)
    def _():
        pool_acc[...] = jnp.full((_G, _H), -jnp.inf, jnp.float32)

    deg = dp_ref[0, :, 0:1] + dp_ref[1, :, 0:1] + 1.0
    dis = lax.rsqrt(deg)
    h = dis * (acc_ref[0] + acc_ref[1] + y_ref[...]) + b1_ref[...][None, :]
    h = jnp.maximum(h, 0.0)
    bb = b_ref[...]
    for g in range(_G):
        m = jnp.where(bb == g, h, -jnp.inf)
        pool_acc[g, :] = jnp.maximum(pool_acc[g, :], jnp.max(m, axis=0))

    @pl.when(i == pl.num_programs(0) - 1)
    def _():
        pool = pool_acc[...]
        pool_ref[...] = pool
        logits_ref[...] = (
            jnp.dot(pool, w2_ref[...], preferred_element_type=jnp.float32)
            + b2_ref[...][None, :]
        )


_epi_call = pl.pallas_call(
    _epi_body,
    grid=(_N // _RB,),
    in_specs=[
        pl.BlockSpec((_NC, _RB, _LANES), lambda i: (0, i, 0)),
        pl.BlockSpec((_NC, _RB, _H), lambda i: (0, i, 0)),
        pl.BlockSpec((_RB, _H), lambda i: (i, 0)),
        pl.BlockSpec((_RB, 1), lambda i: (i, 0)),
        pl.BlockSpec((_H,), lambda i: (0,)),
        pl.BlockSpec((_H, _OUT), lambda i: (0, 0)),
        pl.BlockSpec((_OUT,), lambda i: (0,)),
    ],
    out_specs=[
        pl.BlockSpec((_G, _OUT), lambda i: (0, 0)),
        pl.BlockSpec((_G, _H), lambda i: (0, 0)),
    ],
    out_shape=[
        jax.ShapeDtypeStruct((_G, _OUT), jnp.float32),
        jax.ShapeDtypeStruct((_G, _H), jnp.float32),
    ],
    scratch_shapes=[pltpu.VMEM((_G, _H), jnp.float32)],
)


def kernel(x, edge_index, batch, W1, b1, W2, b2):
    row = edge_index[0]
    col = edge_index[1]
    deg_part = _deg_kernel(col)
    xw = _xw_call(x, W1)
    y = _scale_call(deg_part, xw)
    acc = _gs_kernel(row, col, y)
    logits, x_pool = _epi_call(deg_part, acc, y, batch.reshape(_N, 1),
                               b1, W2, b2)
    return (logits, x_pool)


# speedup vs baseline: 16.5702x; 16.5702x over previous
import functools

import jax
import jax.numpy as jnp
from jax import lax
from jax.experimental import pallas as pl
from jax.experimental.pallas import tpu as pltpu
from jax.experimental.pallas import tpu_sc as plsc

_N = 10000
_E = 320000
_D = 128
_H = 128
_OUT = 10
_G = 32

_NC = 2
_NS = 16
_LANES = 16
_NW = _NC * _NS

_E_TILE = _E // _NW
_CHUNK = 80
_NPAD = 10240
_RPS = _NPAD // _NS
_ZROWS = 128

def _deg_body(col_hbm, deg_hbm, idx_v, ones_v, deg_sh):
    cid = lax.axis_index("c")
    sid = lax.axis_index("s")

    @pl.loop(0, _CHUNK, step=_LANES)
    def _(i):
        ones_v[pl.ds(i, _LANES)] = jnp.zeros((_LANES,), jnp.float32)

    @pl.loop(0, _RPS, step=_CHUNK)
    def _(r):
        pltpu.sync_copy(ones_v, deg_sh.at[pl.ds(sid * _RPS + r, _CHUNK)])

    @pl.loop(0, _CHUNK, step=_LANES)
    def _(i):
        ones_v[pl.ds(i, _LANES)] = jnp.full((_LANES,), 1.0, jnp.float32)

    plsc.subcore_barrier()

    base = (cid * _NS + sid) * _E_TILE

    @pl.loop(0, _E_TILE, step=_CHUNK)
    def _(off):
        pltpu.sync_copy(col_hbm.at[pl.ds(base + off, _CHUNK)], idx_v)
        pltpu.sync_copy(ones_v, deg_sh.at[idx_v], add=True)

    plsc.subcore_barrier()
    pltpu.sync_copy(
        deg_sh.at[pl.ds(sid * _RPS, _RPS)],
        deg_hbm.at[cid, pl.ds(sid * _RPS, _RPS)],
    )


def _gs_body(row_hbm, col_hbm, y_hbm, acc_hbm,
             ridx_v, cidx_v, rows_v, zeros_v, acc_sh, sem):
    cid = lax.axis_index("c")
    sid = lax.axis_index("s")

    @pl.loop(0, _ZROWS)
    def _(i):
        @pl.loop(0, _H, step=_LANES)
        def _(j):
            zeros_v[i, pl.ds(j, _LANES)] = jnp.zeros((_LANES,), jnp.float32)

    @pl.loop(0, _RPS, step=_ZROWS)
    def _(r):
        pltpu.sync_copy(zeros_v, acc_sh.at[pl.ds(sid * _RPS + r, _ZROWS)])

    plsc.subcore_barrier()

    base = (cid * _NS + sid) * _E_TILE

    @pl.loop(0, _E_TILE, step=_CHUNK)
    def _(off):
        pltpu.sync_copy(row_hbm.at[pl.ds(base + off, _CHUNK)], ridx_v)
        pltpu.sync_copy(col_hbm.at[pl.ds(base + off, _CHUNK)], cidx_v)
        pltpu.async_copy(y_hbm.at[ridx_v], rows_v, sem).wait()
        pltpu.sync_copy(rows_v, acc_sh.at[cidx_v], add=True)

    plsc.subcore_barrier()
    pltpu.sync_copy(
        acc_sh.at[pl.ds(sid * _RPS, _RPS)],
        acc_hbm.at[cid, pl.ds(sid * _RPS, _RPS)],
    )


@functools.lru_cache(maxsize=None)
def _sc_kernels():
    mesh = plsc.VectorSubcoreMesh(core_axis_name="c", subcore_axis_name="s",
                                  num_cores=_NC, num_subcores=_NS)
    deg_kernel = pl.kernel(
        _deg_body,
        out_type=jax.ShapeDtypeStruct((_NC, _NPAD), jnp.float32),
        mesh=mesh,
        scratch_types=[
            pltpu.VMEM((_CHUNK,), jnp.int32),
            pltpu.VMEM((_CHUNK,), jnp.float32),
            pltpu.VMEM_SHARED((_NPAD,), jnp.float32),
        ],
    )
    gs_kernel = pl.kernel(
        _gs_body,
        out_type=jax.ShapeDtypeStruct((_NC, _NPAD, _H), jnp.float32),
        mesh=mesh,
        scratch_types=[
            pltpu.VMEM((_CHUNK,), jnp.int32),
            pltpu.VMEM((_CHUNK,), jnp.int32),
            pltpu.VMEM((_CHUNK, _H), jnp.float32),
            pltpu.VMEM((_ZROWS, _H), jnp.float32),
            pltpu.VMEM_SHARED((_NPAD, _H), jnp.float32),
            pltpu.SemaphoreType.DMA,
        ],
    )
    return deg_kernel, gs_kernel


_RB = 1000


def _xw_body(x_ref, w_ref, o_ref):
    o_ref[...] = jnp.dot(x_ref[...], w_ref[...],
                         preferred_element_type=jnp.float32)


_xw_call = pl.pallas_call(
    _xw_body,
    grid=(_N // _RB,),
    in_specs=[
        pl.BlockSpec((_RB, _D), lambda i: (i, 0)),
        pl.BlockSpec((_D, _H), lambda i: (0, 0)),
    ],
    out_specs=pl.BlockSpec((_RB, _H), lambda i: (i, 0)),
    out_shape=jax.ShapeDtypeStruct((_N, _H), jnp.float32),
)


def _scale_body(dp_ref, xw_ref, y_ref):
    deg = dp_ref[0] + dp_ref[1] + 1.0
    y_ref[...] = lax.rsqrt(deg) * xw_ref[...]


_scale_call = pl.pallas_call(
    _scale_body,
    grid=(_N // _RB,),
    in_specs=[
        pl.BlockSpec((_NC, _RB, 1), lambda i: (0, i, 0)),
        pl.BlockSpec((_RB, _H), lambda i: (i, 0)),
    ],
    out_specs=pl.BlockSpec((_RB, _H), lambda i: (i, 0)),
    out_shape=jax.ShapeDtypeStruct((_N, _H), jnp.float32),
)


def _epi_body(dp_ref, acc_ref, y_ref, b_ref, b1_ref, w2_ref, b2_ref,
              logits_ref, pool_ref, pool_acc):
    i = pl.program_id(0)

    @pl.when(i == 0)
    def _():
        pool_acc[...] = jnp.full((_G, _H), -jnp.inf, jnp.float32)

    deg = dp_ref[0] + dp_ref[1] + 1.0
    dis = lax.rsqrt(deg)
    h = dis * (acc_ref[0] + acc_ref[1] + y_ref[...]) + b1_ref[...][None, :]
    h = jnp.maximum(h, 0.0)
    bb = b_ref[...]
    for g in range(_G):
        m = jnp.where(bb == g, h, -jnp.inf)
        pool_acc[g, :] = jnp.maximum(pool_acc[g, :], jnp.max(m, axis=0))

    @pl.when(i == pl.num_programs(0) - 1)
    def _():
        pool = pool_acc[...]
        pool_ref[...] = pool
        logits_ref[...] = (
            jnp.dot(pool, w2_ref[...], preferred_element_type=jnp.float32)
            + b2_ref[...][None, :]
        )


_epi_call = pl.pallas_call(
    _epi_body,
    grid=(_N // _RB,),
    in_specs=[
        pl.BlockSpec((_NC, _RB, 1), lambda i: (0, i, 0)),
        pl.BlockSpec((_NC, _RB, _H), lambda i: (0, i, 0)),
        pl.BlockSpec((_RB, _H), lambda i: (i, 0)),
        pl.BlockSpec((_RB, 1), lambda i: (i, 0)),
        pl.BlockSpec((_H,), lambda i: (0,)),
        pl.BlockSpec((_H, _OUT), lambda i: (0, 0)),
        pl.BlockSpec((_OUT,), lambda i: (0,)),
    ],
    out_specs=[
        pl.BlockSpec((_G, _OUT), lambda i: (0, 0)),
        pl.BlockSpec((_G, _H), lambda i: (0, 0)),
    ],
    out_shape=[
        jax.ShapeDtypeStruct((_G, _OUT), jnp.float32),
        jax.ShapeDtypeStruct((_G, _H), jnp.float32),
    ],
    scratch_shapes=[pltpu.VMEM((_G, _H), jnp.float32)],
)


def kernel(x, edge_index, batch, W1, b1, W2, b2):
    row = edge_index[0]
    col = edge_index[1]
    deg_kernel, gs_kernel = _sc_kernels()
    deg_part = deg_kernel(col).reshape(_NC, _NPAD, 1)
    xw = _xw_call(x, W1)
    y = _scale_call(deg_part, xw)
    acc = gs_kernel(row, col, y)
    logits, x_pool = _epi_call(deg_part, acc, y, batch.reshape(_N, 1),
                               b1, W2, b2)
    return (logits, x_pool)


# speedup vs baseline: 22.5745x; 1.3624x over previous
import functools

import jax
import jax.numpy as jnp
from jax import lax
from jax.experimental import pallas as pl
from jax.experimental.pallas import tpu as pltpu
from jax.experimental.pallas import tpu_sc as plsc

_N = 10000
_E = 320000
_D = 128
_H = 128
_OUT = 10
_G = 32

_NC = 2
_NS = 16
_LANES = 16
_NW = _NC * _NS

_E_TILE = _E // _NW
_CHUNK = 80
_NPAD = 10240
_RPS = _NPAD // _NS
_ZROWS = 128

def _deg_body(col_hbm, deg_hbm, idx_v, ones_v, deg_sh):
    cid = lax.axis_index("c")
    sid = lax.axis_index("s")

    @pl.loop(0, _CHUNK, step=_LANES)
    def _(i):
        ones_v[pl.ds(i, _LANES)] = jnp.zeros((_LANES,), jnp.float32)

    @pl.loop(0, _RPS, step=_CHUNK)
    def _(r):
        pltpu.sync_copy(ones_v, deg_sh.at[pl.ds(sid * _RPS + r, _CHUNK)])

    @pl.loop(0, _CHUNK, step=_LANES)
    def _(i):
        ones_v[pl.ds(i, _LANES)] = jnp.full((_LANES,), 1.0, jnp.float32)

    plsc.subcore_barrier()

    base = (cid * _NS + sid) * _E_TILE

    @pl.loop(0, _E_TILE, step=_CHUNK)
    def _(off):
        pltpu.sync_copy(col_hbm.at[pl.ds(base + off, _CHUNK)], idx_v)
        pltpu.sync_copy(ones_v, deg_sh.at[idx_v], add=True)

    plsc.subcore_barrier()
    pltpu.sync_copy(
        deg_sh.at[pl.ds(sid * _RPS, _RPS)],
        deg_hbm.at[cid, pl.ds(sid * _RPS, _RPS)],
    )


def _gs_body(row_hbm, col_hbm, y_hbm, acc_hbm,
             ridx_v, cidx_v, rows_v, zeros_v, acc_sh, sg0, sg1):
    cid = lax.axis_index("c")
    sid = lax.axis_index("s")
    gsems = (sg0, sg1)

    @pl.loop(0, _ZROWS)
    def _(i):
        @pl.loop(0, _H, step=_LANES)
        def _(j):
            zeros_v[i, pl.ds(j, _LANES)] = jnp.zeros((_LANES,), jnp.float32)

    @pl.loop(0, _RPS, step=_ZROWS)
    def _(r):
        pltpu.sync_copy(zeros_v, acc_sh.at[pl.ds(sid * _RPS + r, _ZROWS)])

    plsc.subcore_barrier()

    base = (cid * _NS + sid) * _E_TILE

    def load_idx(j, b):
        off = base + j * _CHUNK
        pltpu.sync_copy(row_hbm.at[pl.ds(off, _CHUNK)], ridx_v.at[b])
        pltpu.sync_copy(col_hbm.at[pl.ds(off, _CHUNK)], cidx_v.at[b])

    def gather_start(b):
        pltpu.async_copy(y_hbm.at[ridx_v.at[b]], rows_v.at[b], gsems[b])

    def gather_wait(b):
        pltpu.make_async_copy(y_hbm.at[ridx_v.at[b]], rows_v.at[b],
                              gsems[b]).wait()

    def scatter_sync(b):
        pltpu.sync_copy(rows_v.at[b], acc_sh.at[cidx_v.at[b]], add=True)

    load_idx(0, 0)
    gather_start(0)

    @pl.loop(0, (_E_TILE // _CHUNK - 1) // 2)
    def _(p):
        j1 = 2 * p + 1
        load_idx(j1, 1)
        gather_start(1)
        gather_wait(0)
        scatter_sync(0)
        load_idx(j1 + 1, 0)
        gather_start(0)
        gather_wait(1)
        scatter_sync(1)

    gather_wait(0)
    scatter_sync(0)

    plsc.subcore_barrier()
    pltpu.sync_copy(
        acc_sh.at[pl.ds(sid * _RPS, _RPS)],
        acc_hbm.at[cid, pl.ds(sid * _RPS, _RPS)],
    )


@functools.lru_cache(maxsize=None)
def _sc_kernels():
    mesh = plsc.VectorSubcoreMesh(core_axis_name="c", subcore_axis_name="s",
                                  num_cores=_NC, num_subcores=_NS)
    deg_kernel = pl.kernel(
        _deg_body,
        out_type=jax.ShapeDtypeStruct((_NC, _NPAD), jnp.float32),
        mesh=mesh,
        scratch_types=[
            pltpu.VMEM((_CHUNK,), jnp.int32),
            pltpu.VMEM((_CHUNK,), jnp.float32),
            pltpu.VMEM_SHARED((_NPAD,), jnp.float32),
        ],
    )
    gs_kernel = pl.kernel(
        _gs_body,
        out_type=jax.ShapeDtypeStruct((_NC, _NPAD, _H), jnp.float32),
        mesh=mesh,
        scratch_types=[
            pltpu.VMEM((2, _CHUNK), jnp.int32),
            pltpu.VMEM((2, _CHUNK), jnp.int32),
            pltpu.VMEM((2, _CHUNK, _H), jnp.float32),
            pltpu.VMEM((_ZROWS, _H), jnp.float32),
            pltpu.VMEM_SHARED((_NPAD, _H), jnp.float32),
            pltpu.SemaphoreType.DMA,
            pltpu.SemaphoreType.DMA,
        ],
    )
    return deg_kernel, gs_kernel


_RB = 1000


def _xw_body(x_ref, w_ref, o_ref):
    o_ref[...] = jnp.dot(x_ref[...], w_ref[...],
                         preferred_element_type=jnp.float32)


_xw_call = pl.pallas_call(
    _xw_body,
    grid=(_N // _RB,),
    in_specs=[
        pl.BlockSpec((_RB, _D), lambda i: (i, 0)),
        pl.BlockSpec((_D, _H), lambda i: (0, 0)),
    ],
    out_specs=pl.BlockSpec((_RB, _H), lambda i: (i, 0)),
    out_shape=jax.ShapeDtypeStruct((_N, _H), jnp.float32),
)


def _scale_body(dp_ref, xw_ref, y_ref):
    deg = dp_ref[0] + dp_ref[1] + 1.0
    y_ref[...] = lax.rsqrt(deg) * xw_ref[...]


_scale_call = pl.pallas_call(
    _scale_body,
    grid=(_N // _RB,),
    in_specs=[
        pl.BlockSpec((_NC, _RB, 1), lambda i: (0, i, 0)),
        pl.BlockSpec((_RB, _H), lambda i: (i, 0)),
    ],
    out_specs=pl.BlockSpec((_RB, _H), lambda i: (i, 0)),
    out_shape=jax.ShapeDtypeStruct((_N, _H), jnp.float32),
)


def _epi_body(dp_ref, acc_ref, y_ref, b_ref, b1_ref, w2_ref, b2_ref,
              logits_ref, pool_ref, pool_acc):
    i = pl.program_id(0)

    @pl.when(i == 0)
    def _():
        pool_acc[...] = jnp.full((_G, _H), -jnp.inf, jnp.float32)

    deg = dp_ref[0] + dp_ref[1] + 1.0
    dis = lax.rsqrt(deg)
    h = dis * (acc_ref[0] + acc_ref[1] + y_ref[...]) + b1_ref[...][None, :]
    h = jnp.maximum(h, 0.0)
    bb = b_ref[...]
    for g in range(_G):
        m = jnp.where(bb == g, h, -jnp.inf)
        pool_acc[g, :] = jnp.maximum(pool_acc[g, :], jnp.max(m, axis=0))

    @pl.when(i == pl.num_programs(0) - 1)
    def _():
        pool = pool_acc[...]
        pool_ref[...] = pool
        logits_ref[...] = (
            jnp.dot(pool, w2_ref[...], preferred_element_type=jnp.float32)
            + b2_ref[...][None, :]
        )


_epi_call = pl.pallas_call(
    _epi_body,
    grid=(_N // _RB,),
    in_specs=[
        pl.BlockSpec((_NC, _RB, 1), lambda i: (0, i, 0)),
        pl.BlockSpec((_NC, _RB, _H), lambda i: (0, i, 0)),
        pl.BlockSpec((_RB, _H), lambda i: (i, 0)),
        pl.BlockSpec((_RB, 1), lambda i: (i, 0)),
        pl.BlockSpec((_H,), lambda i: (0,)),
        pl.BlockSpec((_H, _OUT), lambda i: (0, 0)),
        pl.BlockSpec((_OUT,), lambda i: (0,)),
    ],
    out_specs=[
        pl.BlockSpec((_G, _OUT), lambda i: (0, 0)),
        pl.BlockSpec((_G, _H), lambda i: (0, 0)),
    ],
    out_shape=[
        jax.ShapeDtypeStruct((_G, _OUT), jnp.float32),
        jax.ShapeDtypeStruct((_G, _H), jnp.float32),
    ],
    scratch_shapes=[pltpu.VMEM((_G, _H), jnp.float32)],
)


def kernel(x, edge_index, batch, W1, b1, W2, b2):
    row = edge_index[0]
    col = edge_index[1]
    deg_kernel, gs_kernel = _sc_kernels()
    deg_part = deg_kernel(col).reshape(_NC, _NPAD, 1)
    xw = _xw_call(x, W1)
    y = _scale_call(deg_part, xw)
    acc = gs_kernel(row, col, y)
    logits, x_pool = _epi_call(deg_part, acc, y, batch.reshape(_N, 1),
                               b1, W2, b2)
    return (logits, x_pool)


# speedup vs baseline: 27.2242x; 1.2060x over previous
import functools

import jax
import jax.numpy as jnp
from jax import lax
from jax.experimental import pallas as pl
from jax.experimental.pallas import tpu as pltpu
from jax.experimental.pallas import tpu_sc as plsc

_N = 10000
_E = 320000
_D = 128
_H = 128
_OUT = 10
_G = 32

_NC = 2
_NS = 16
_LANES = 16
_NW = _NC * _NS

_E_TILE = _E // _NW
_CHUNK = 80
_NPAD = 10240
_RPS = _NPAD // _NS
_ZROWS = 128

def _deg_body(col_hbm, deg_hbm, idx_v, ones_v, deg_sh, sd0, sd1):
    cid = lax.axis_index("c")
    sid = lax.axis_index("s")
    ssems = (sd0, sd1)

    @pl.loop(0, _CHUNK, step=_LANES)
    def _(i):
        ones_v[pl.ds(i, _LANES)] = jnp.zeros((_LANES,), jnp.float32)

    @pl.loop(0, _RPS, step=_CHUNK)
    def _(r):
        pltpu.sync_copy(ones_v, deg_sh.at[pl.ds(sid * _RPS + r, _CHUNK)])

    @pl.loop(0, _CHUNK, step=_LANES)
    def _(i):
        ones_v[pl.ds(i, _LANES)] = jnp.full((_LANES,), 1.0, jnp.float32)

    plsc.subcore_barrier()

    base = (cid * _NS + sid) * _E_TILE

    def load_idx(j, b):
        pltpu.sync_copy(col_hbm.at[pl.ds(base + j * _CHUNK, _CHUNK)],
                        idx_v.at[b])

    def sc_start(b):
        pltpu.async_copy(ones_v, deg_sh.at[idx_v.at[b]], ssems[b], add=True)

    def sc_wait(b):
        pltpu.make_async_copy(ones_v, deg_sh.at[idx_v.at[b]], ssems[b]).wait()

    load_idx(0, 0)
    sc_start(0)

    @pl.loop(0, (_E_TILE // _CHUNK - 1) // 2)
    def _(p):
        j1 = 2 * p + 1
        load_idx(j1, 1)
        sc_start(1)
        sc_wait(0)
        load_idx(j1 + 1, 0)
        sc_start(0)
        sc_wait(1)

    sc_wait(0)

    plsc.subcore_barrier()
    pltpu.sync_copy(
        deg_sh.at[pl.ds(sid * _RPS, _RPS)],
        deg_hbm.at[cid, pl.ds(sid * _RPS, _RPS)],
    )


def _gs_body(row_hbm, col_hbm, y_hbm, acc_hbm,
             ridx_v, cidx_v, rows_v, acc_sh,
             sg0, sg1, sg2, sg3, ss0, ss1, ss2, ss3):
    cid = lax.axis_index("c")
    sid = lax.axis_index("s")
    gsems = (sg0, sg1, sg2, sg3)
    ssems = (ss0, ss1, ss2, ss3)

    @pl.loop(0, _CHUNK)
    def _(i):
        @pl.loop(0, _H, step=_LANES)
        def _(j):
            rows_v[0, i, pl.ds(j, _LANES)] = jnp.zeros((_LANES,), jnp.float32)

    @pl.loop(0, _RPS, step=_CHUNK)
    def _(r):
        pltpu.sync_copy(rows_v.at[0], acc_sh.at[pl.ds(sid * _RPS + r, _CHUNK)])

    plsc.subcore_barrier()

    base = (cid * _NS + sid) * _E_TILE

    def load_idx(j, b):
        off = base + j * _CHUNK
        pltpu.sync_copy(row_hbm.at[pl.ds(off, _CHUNK)], ridx_v.at[b])
        pltpu.sync_copy(col_hbm.at[pl.ds(off, _CHUNK)], cidx_v.at[b])

    def gather_start(b):
        pltpu.async_copy(y_hbm.at[ridx_v.at[b]], rows_v.at[b], gsems[b])

    def gather_wait(b):
        pltpu.make_async_copy(y_hbm.at[ridx_v.at[b]], rows_v.at[b],
                              gsems[b]).wait()

    def scatter_start(b):
        pltpu.async_copy(rows_v.at[b], acc_sh.at[cidx_v.at[b]], ssems[b],
                         add=True)

    def scatter_wait(b):
        pltpu.make_async_copy(rows_v.at[b], acc_sh.at[cidx_v.at[b]],
                              ssems[b]).wait()

    for j in range(4):
        load_idx(j, j)
        gather_start(j)
        if j >= 2:
            gather_wait(j - 2)
            scatter_start(j - 2)

    @pl.loop(4, _E_TILE // _CHUNK - 1, step=4)
    def _(j0):
        for b in range(4):
            j = j0 + b
            scatter_wait(b)
            load_idx(j, b)
            gather_start(b)
            pb = (b + 2) % 4
            gather_wait(pb)
            scatter_start(pb)

    scatter_wait(0)
    load_idx(_E_TILE // _CHUNK - 1, 0)
    gather_start(0)
    gather_wait(2)
    scatter_start(2)
    gather_wait(3)
    scatter_start(3)
    gather_wait(0)
    scatter_start(0)
    scatter_wait(1)
    scatter_wait(2)
    scatter_wait(3)
    scatter_wait(0)

    plsc.subcore_barrier()
    pltpu.sync_copy(
        acc_sh.at[pl.ds(sid * _RPS, _RPS)],
        acc_hbm.at[cid, pl.ds(sid * _RPS, _RPS)],
    )


@functools.lru_cache(maxsize=None)
def _sc_kernels():
    mesh = plsc.VectorSubcoreMesh(core_axis_name="c", subcore_axis_name="s",
                                  num_cores=_NC, num_subcores=_NS)
    deg_kernel = pl.kernel(
        _deg_body,
        out_type=jax.ShapeDtypeStruct((_NC, _NPAD), jnp.float32),
        mesh=mesh,
        scratch_types=[
            pltpu.VMEM((2, _CHUNK), jnp.int32),
            pltpu.VMEM((_CHUNK,), jnp.float32),
            pltpu.VMEM_SHARED((_NPAD,), jnp.float32),
            pltpu.SemaphoreType.DMA,
            pltpu.SemaphoreType.DMA,
        ],
    )
    gs_kernel = pl.kernel(
        _gs_body,
        out_type=jax.ShapeDtypeStruct((_NC, _NPAD, _H), jnp.float32),
        mesh=mesh,
        scratch_types=[
            pltpu.VMEM((4, _CHUNK), jnp.int32),
            pltpu.VMEM((4, _CHUNK), jnp.int32),
            pltpu.VMEM((4, _CHUNK, _H), jnp.float32),
            pltpu.VMEM_SHARED((_NPAD, _H), jnp.float32),
        ] + [pltpu.SemaphoreType.DMA] * 8,
    )
    return deg_kernel, gs_kernel


_RB = 1000


def _xw_body(x_ref, w_ref, o_ref):
    o_ref[...] = jnp.dot(x_ref[...], w_ref[...],
                         preferred_element_type=jnp.float32)


_xw_call = pl.pallas_call(
    _xw_body,
    grid=(_N // _RB,),
    in_specs=[
        pl.BlockSpec((_RB, _D), lambda i: (i, 0)),
        pl.BlockSpec((_D, _H), lambda i: (0, 0)),
    ],
    out_specs=pl.BlockSpec((_RB, _H), lambda i: (i, 0)),
    out_shape=jax.ShapeDtypeStruct((_N, _H), jnp.float32),
)


def _scale_body(dp_ref, xw_ref, y_ref):
    deg = dp_ref[0] + dp_ref[1] + 1.0
    y_ref[...] = lax.rsqrt(deg) * xw_ref[...]


_scale_call = pl.pallas_call(
    _scale_body,
    grid=(_N // _RB,),
    in_specs=[
        pl.BlockSpec((_NC, _RB, 1), lambda i: (0, i, 0)),
        pl.BlockSpec((_RB, _H), lambda i: (i, 0)),
    ],
    out_specs=pl.BlockSpec((_RB, _H), lambda i: (i, 0)),
    out_shape=jax.ShapeDtypeStruct((_N, _H), jnp.float32),
)


def _epi_body(dp_ref, acc_ref, y_ref, b_ref, b1_ref, w2_ref, b2_ref,
              logits_ref, pool_ref, pool_acc):
    i = pl.program_id(0)

    @pl.when(i == 0)
    def _():
        pool_acc[...] = jnp.full((_G, _H), -jnp.inf, jnp.float32)

    deg = dp_ref[0] + dp_ref[1] + 1.0
    dis = lax.rsqrt(deg)
    h = dis * (acc_ref[0] + acc_ref[1] + y_ref[...]) + b1_ref[...][None, :]
    h = jnp.maximum(h, 0.0)
    bb = b_ref[...]
    for g in range(_G):
        m = jnp.where(bb == g, h, -jnp.inf)
        pool_acc[g, :] = jnp.maximum(pool_acc[g, :], jnp.max(m, axis=0))

    @pl.when(i == pl.num_programs(0) - 1)
    def _():
        pool = pool_acc[...]
        pool_ref[...] = pool
        logits_ref[...] = (
            jnp.dot(pool, w2_ref[...], preferred_element_type=jnp.float32)
            + b2_ref[...][None, :]
        )


_epi_call = pl.pallas_call(
    _epi_body,
    grid=(_N // _RB,),
    in_specs=[
        pl.BlockSpec((_NC, _RB, 1), lambda i: (0, i, 0)),
        pl.BlockSpec((_NC, _RB, _H), lambda i: (0, i, 0)),
        pl.BlockSpec((_RB, _H), lambda i: (i, 0)),
        pl.BlockSpec((_RB, 1), lambda i: (i, 0)),
        pl.BlockSpec((_H,), lambda i: (0,)),
        pl.BlockSpec((_H, _OUT), lambda i: (0, 0)),
        pl.BlockSpec((_OUT,), lambda i: (0,)),
    ],
    out_specs=[
        pl.BlockSpec((_G, _OUT), lambda i: (0, 0)),
        pl.BlockSpec((_G, _H), lambda i: (0, 0)),
    ],
    out_shape=[
        jax.ShapeDtypeStruct((_G, _OUT), jnp.float32),
        jax.ShapeDtypeStruct((_G, _H), jnp.float32),
    ],
    scratch_shapes=[pltpu.VMEM((_G, _H), jnp.float32)],
)


def kernel(x, edge_index, batch, W1, b1, W2, b2):
    row = edge_index[0]
    col = edge_index[1]
    deg_kernel, gs_kernel = _sc_kernels()
    deg_part = deg_kernel(col).reshape(_NC, _NPAD, 1)
    xw = _xw_call(x, W1)
    y = _scale_call(deg_part, xw)
    acc = gs_kernel(row, col, y)
    logits, x_pool = _epi_call(deg_part, acc, y, batch.reshape(_N, 1),
                               b1, W2, b2)
    return (logits, x_pool)


# speedup vs baseline: 35.7093x; 1.3117x over previous
import functools

import jax
import jax.numpy as jnp
from jax import lax
from jax.experimental import pallas as pl
from jax.experimental.pallas import tpu as pltpu
from jax.experimental.pallas import tpu_sc as plsc

_N = 10000
_E = 320000
_D = 128
_H = 128
_OUT = 10
_G = 32

_NC = 2
_NS = 16
_LANES = 16
_NW = _NC * _NS

_E_TILE = _E // _NW
_CHUNK = 80
_NCH = _E_TILE // _CHUNK
_NPAD = 10240
_RPS = _NPAD // _NS


def _deg_body(ei_hbm, deg_hbm, idx_v, ones_v, deg_sh, *sems):
    cid = lax.axis_index("c")
    sid = lax.axis_index("s")
    isems = sems[0:8]
    ssems = sems[8:16]

    @pl.loop(0, _CHUNK, step=_LANES)
    def _(i):
        ones_v[pl.ds(i, _LANES)] = jnp.zeros((_LANES,), jnp.float32)

    @pl.loop(0, _RPS, step=_CHUNK)
    def _(r):
        pltpu.sync_copy(ones_v, deg_sh.at[pl.ds(sid * _RPS + r, _CHUNK)])

    @pl.loop(0, _CHUNK, step=_LANES)
    def _(i):
        ones_v[pl.ds(i, _LANES)] = jnp.full((_LANES,), 1.0, jnp.float32)

    plsc.subcore_barrier()

    cbase = (cid * _NS + sid) * _NCH

    def i_start(j, s):
        pltpu.async_copy(ei_hbm.at[cbase + j], idx_v.at[s], isems[s])

    def i_wait(s):
        pltpu.make_async_copy(ei_hbm.at[cbase], idx_v.at[s], isems[s]).wait()

    def s_start(s):
        pltpu.async_copy(ones_v, deg_sh.at[idx_v.at[s, 1]], ssems[s],
                         add=True)

    def s_wait(s):
        pltpu.make_async_copy(ones_v, deg_sh.at[idx_v.at[0, 1]],
                              ssems[s]).wait()

    for s in range(4):
        i_start(s, s)
    for j in range(4):
        i_start(j + 4, j + 4)
        i_wait(j)
        s_start(j)
    for j in range(4, 8):
        s_wait((j + 4) % 8)
        i_start(j + 4, (j + 4) % 8)
        i_wait(j)
        s_start(j)

    @pl.loop(8, _NCH - 5, step=8)
    def _(j0):
        for k in range(8):
            j = j0 + k
            s_wait((k + 4) % 8)
            i_start(j + 4, (k + 4) % 8)
            i_wait(k)
            s_start(k)

    s_wait(4)
    i_start(_NCH - 1, 4)
    i_wait(0)
    s_start(0)
    for k in range(1, 4):
        s_wait(k + 4)
        i_wait(k)
        s_start(k)
    s_wait(0)
    i_wait(4)
    s_start(4)
    for s in range(1, 5):
        s_wait(s)

    plsc.subcore_barrier()
    pltpu.sync_copy(
        deg_sh.at[pl.ds(sid * _RPS, _RPS)],
        deg_hbm.at[cid, pl.ds(sid * _RPS, _RPS)],
    )


def _gs_body(ei_hbm, y_hbm, acc_hbm, idx_v, rows_v, acc_sh, *sems):
    cid = lax.axis_index("c")
    sid = lax.axis_index("s")
    gsems = sems[0:4]
    ssems = sems[4:8]
    isems = sems[8:16]

    @pl.loop(0, _CHUNK)
    def _(i):
        @pl.loop(0, _H, step=_LANES)
        def _(j):
            rows_v[0, i, pl.ds(j, _LANES)] = jnp.zeros((_LANES,), jnp.float32)

    @pl.loop(0, _RPS, step=_CHUNK)
    def _(r):
        pltpu.sync_copy(rows_v.at[0], acc_sh.at[pl.ds(sid * _RPS + r, _CHUNK)])

    plsc.subcore_barrier()

    cbase = (cid * _NS + sid) * _NCH

    def i_start(j, s):
        pltpu.async_copy(ei_hbm.at[cbase + j], idx_v.at[s], isems[s])

    def i_wait(s):
        pltpu.make_async_copy(ei_hbm.at[cbase], idx_v.at[s], isems[s]).wait()

    def g_start(b, s):
        pltpu.async_copy(y_hbm.at[idx_v.at[s, 0]], rows_v.at[b], gsems[b])

    def g_wait(b, s):
        pltpu.make_async_copy(y_hbm.at[idx_v.at[s, 0]], rows_v.at[b],
                              gsems[b]).wait()

    def s_start(b, s):
        pltpu.async_copy(rows_v.at[b], acc_sh.at[idx_v.at[s, 1]], ssems[b],
                         add=True)

    def s_wait(b):
        pltpu.make_async_copy(rows_v.at[b], acc_sh.at[idx_v.at[0, 1]],
                              ssems[b]).wait()

    for s in range(4):
        i_start(s, s)
    for j in range(4):
        i_start(j + 4, j + 4)
        i_wait(j)
        g_start(j, j)
        if j >= 2:
            g_wait(j - 2, j - 2)
            s_start(j - 2, j - 2)
    for j in range(4, 8):
        s_wait(j % 4)
        i_start(j + 4, (j + 4) % 8)
        i_wait(j)
        g_start(j % 4, j)
        g_wait((j - 2) % 4, j - 2)
        s_start((j - 2) % 4, j - 2)

    @pl.loop(8, _NCH - 5, step=8)
    def _(j0):
        for k in range(8):
            j = j0 + k
            s_wait(k % 4)
            i_start(j + 4, (k + 4) % 8)
            i_wait(k)
            g_start(k % 4, k)
            g_wait((k - 2) % 4, (k - 2) % 8)
            s_start((k - 2) % 4, (k - 2) % 8)

    s_wait(0)
    i_start(_NCH - 1, 4)
    i_wait(0)
    g_start(0, 0)
    g_wait(2, 6)
    s_start(2, 6)
    for k in range(1, 4):
        s_wait(k)
        i_wait(k)
        g_start(k, k)
        g_wait((k + 2) % 4, (k + 6) % 8)
        s_start((k + 2) % 4, (k + 6) % 8)
    s_wait(0)
    i_wait(4)
    g_start(0, 4)
    g_wait(2, 2)
    s_start(2, 2)
    g_wait(3, 3)
    s_start(3, 3)
    g_wait(0, 4)
    s_start(0, 4)
    for b in (1, 2, 3, 0):
        s_wait(b)

    plsc.subcore_barrier()
    pltpu.sync_copy(
        acc_sh.at[pl.ds(sid * _RPS, _RPS)],
        acc_hbm.at[cid, pl.ds(sid * _RPS, _RPS)],
    )


@functools.lru_cache(maxsize=None)
def _sc_kernels():
    mesh = plsc.VectorSubcoreMesh(core_axis_name="c", subcore_axis_name="s",
                                  num_cores=_NC, num_subcores=_NS)
    deg_kernel = pl.kernel(
        _deg_body,
        out_type=jax.ShapeDtypeStruct((_NC, _NPAD), jnp.float32),
        mesh=mesh,
        scratch_types=[
            pltpu.VMEM((8, 2, _CHUNK), jnp.int32),
            pltpu.VMEM((_CHUNK,), jnp.float32),
            pltpu.VMEM_SHARED((_NPAD,), jnp.float32),
        ] + [pltpu.SemaphoreType.DMA] * 16,
    )
    gs_kernel = pl.kernel(
        _gs_body,
        out_type=jax.ShapeDtypeStruct((_NC, _NPAD, _H), jnp.float32),
        mesh=mesh,
        scratch_types=[
            pltpu.VMEM((8, 2, _CHUNK), jnp.int32),
            pltpu.VMEM((4, _CHUNK, _H), jnp.float32),
            pltpu.VMEM_SHARED((_NPAD, _H), jnp.float32),
        ] + [pltpu.SemaphoreType.DMA] * 16,
    )
    return deg_kernel, gs_kernel


_RB = 1000


def _xw_body(x_ref, w_ref, o_ref):
    o_ref[...] = jnp.dot(x_ref[...], w_ref[...],
                         preferred_element_type=jnp.float32)


_xw_call = pl.pallas_call(
    _xw_body,
    grid=(_N // _RB,),
    in_specs=[
        pl.BlockSpec((_RB, _D), lambda i: (i, 0)),
        pl.BlockSpec((_D, _H), lambda i: (0, 0)),
    ],
    out_specs=pl.BlockSpec((_RB, _H), lambda i: (i, 0)),
    out_shape=jax.ShapeDtypeStruct((_N, _H), jnp.float32),
)


def _scale_body(dp_ref, xw_ref, y_ref):
    deg = dp_ref[0] + dp_ref[1] + 1.0
    y_ref[...] = lax.rsqrt(deg) * xw_ref[...]


_scale_call = pl.pallas_call(
    _scale_body,
    grid=(_N // _RB,),
    in_specs=[
        pl.BlockSpec((_NC, _RB, 1), lambda i: (0, i, 0)),
        pl.BlockSpec((_RB, _H), lambda i: (i, 0)),
    ],
    out_specs=pl.BlockSpec((_RB, _H), lambda i: (i, 0)),
    out_shape=jax.ShapeDtypeStruct((_N, _H), jnp.float32),
)


def _epi_body(dp_ref, acc_ref, y_ref, b_ref, b1_ref, w2_ref, b2_ref,
              logits_ref, pool_ref, pool_acc):
    i = pl.program_id(0)

    @pl.when(i == 0)
    def _():
        pool_acc[...] = jnp.full((_G, _H), -jnp.inf, jnp.float32)

    deg = dp_ref[0] + dp_ref[1] + 1.0
    dis = lax.rsqrt(deg)
    h = dis * (acc_ref[0] + acc_ref[1] + y_ref[...]) + b1_ref[...][None, :]
    h = jnp.maximum(h, 0.0)
    bb = b_ref[...]
    for g in range(_G):
        m = jnp.where(bb == g, h, -jnp.inf)
        pool_acc[g, :] = jnp.maximum(pool_acc[g, :], jnp.max(m, axis=0))

    @pl.when(i == pl.num_programs(0) - 1)
    def _():
        pool = pool_acc[...]
        pool_ref[...] = pool
        logits_ref[...] = (
            jnp.dot(pool, w2_ref[...], preferred_element_type=jnp.float32)
            + b2_ref[...][None, :]
        )


_epi_call = pl.pallas_call(
    _epi_body,
    grid=(_N // _RB,),
    in_specs=[
        pl.BlockSpec((_NC, _RB, 1), lambda i: (0, i, 0)),
        pl.BlockSpec((_NC, _RB, _H), lambda i: (0, i, 0)),
        pl.BlockSpec((_RB, _H), lambda i: (i, 0)),
        pl.BlockSpec((_RB, 1), lambda i: (i, 0)),
        pl.BlockSpec((_H,), lambda i: (0,)),
        pl.BlockSpec((_H, _OUT), lambda i: (0, 0)),
        pl.BlockSpec((_OUT,), lambda i: (0,)),
    ],
    out_specs=[
        pl.BlockSpec((_G, _OUT), lambda i: (0, 0)),
        pl.BlockSpec((_G, _H), lambda i: (0, 0)),
    ],
    out_shape=[
        jax.ShapeDtypeStruct((_G, _OUT), jnp.float32),
        jax.ShapeDtypeStruct((_G, _H), jnp.float32),
    ],
    scratch_shapes=[pltpu.VMEM((_G, _H), jnp.float32)],
)


def kernel(x, edge_index, batch, W1, b1, W2, b2):
    ei3 = edge_index.reshape(2, _E // _CHUNK, _CHUNK).transpose(1, 0, 2)
    deg_kernel, gs_kernel = _sc_kernels()
    deg_part = deg_kernel(ei3).reshape(_NC, _NPAD, 1)
    xw = _xw_call(x, W1)
    y = _scale_call(deg_part, xw)
    acc = gs_kernel(ei3, y)
    logits, x_pool = _epi_call(deg_part, acc, y, batch.reshape(_N, 1),
                               b1, W2, b2)
    return (logits, x_pool)


# speedup vs baseline: 41.2045x; 1.1539x over previous
import functools

import jax
import jax.numpy as jnp
from jax import lax
from jax.experimental import pallas as pl
from jax.experimental.pallas import tpu as pltpu
from jax.experimental.pallas import tpu_sc as plsc

_N = 10000
_E = 320000
_D = 128
_H = 128
_OUT = 10
_G = 32

_NC = 2
_NS = 16
_LANES = 16
_NW = _NC * _NS

_E_TILE = _E // _NW
_CHUNK = 80
_NCH = _E_TILE // _CHUNK
_NPAD = 10240
_RPS = _NPAD // _NS


def _deg_body(ei_hbm, deg_hbm, idx_v, ones_v, deg_sh, *sems):
    cid = lax.axis_index("c")
    sid = lax.axis_index("s")
    isems = sems[0:8]
    ssems = sems[8:16]

    @pl.loop(0, _CHUNK, step=_LANES)
    def _(i):
        ones_v[pl.ds(i, _LANES)] = jnp.zeros((_LANES,), jnp.float32)

    @pl.loop(0, _RPS, step=_CHUNK)
    def _(r):
        pltpu.sync_copy(ones_v, deg_sh.at[pl.ds(sid * _RPS + r, _CHUNK)])

    @pl.loop(0, _CHUNK, step=_LANES)
    def _(i):
        ones_v[pl.ds(i, _LANES)] = jnp.full((_LANES,), 1.0, jnp.float32)

    plsc.subcore_barrier()

    cbase = (cid * _NS + sid) * _NCH

    def i_start(j, s):
        pltpu.async_copy(ei_hbm.at[cbase + j], idx_v.at[s], isems[s])

    def i_wait(s):
        pltpu.make_async_copy(ei_hbm.at[cbase], idx_v.at[s], isems[s]).wait()

    def s_start(s):
        pltpu.async_copy(ones_v, deg_sh.at[idx_v.at[s, 1]], ssems[s],
                         add=True)

    def s_wait(s):
        pltpu.make_async_copy(ones_v, deg_sh.at[idx_v.at[0, 1]],
                              ssems[s]).wait()

    for s in range(4):
        i_start(s, s)
    for j in range(4):
        i_start(j + 4, j + 4)
        i_wait(j)
        s_start(j)
    for j in range(4, 8):
        s_wait((j + 4) % 8)
        i_start(j + 4, (j + 4) % 8)
        i_wait(j)
        s_start(j)

    @pl.loop(8, _NCH - 5, step=8)
    def _(j0):
        for k in range(8):
            j = j0 + k
            s_wait((k + 4) % 8)
            i_start(j + 4, (k + 4) % 8)
            i_wait(k)
            s_start(k)

    s_wait(4)
    i_start(_NCH - 1, 4)
    i_wait(0)
    s_start(0)
    for k in range(1, 4):
        s_wait(k + 4)
        i_wait(k)
        s_start(k)
    s_wait(0)
    i_wait(4)
    s_start(4)
    for s in range(1, 5):
        s_wait(s)

    plsc.subcore_barrier()
    pltpu.sync_copy(
        deg_sh.at[pl.ds(sid * _RPS, _RPS)],
        deg_hbm.at[cid, pl.ds(sid * _RPS, _RPS)],
    )


def _gs_body(ei_hbm, y_hbm, acc_hbm, idx_v, rows_v, acc_sh, *sems):
    cid = lax.axis_index("c")
    sid = lax.axis_index("s")
    gsems = sems[0:4]
    ssems = sems[4:8]
    isems = sems[8:16]

    @pl.loop(0, _CHUNK)
    def _(i):
        @pl.loop(0, _H, step=_LANES)
        def _(j):
            rows_v[0, i, pl.ds(j, _LANES)] = jnp.zeros((_LANES,), jnp.float32)

    @pl.loop(0, _RPS, step=_CHUNK)
    def _(r):
        pltpu.sync_copy(rows_v.at[0], acc_sh.at[pl.ds(sid * _RPS + r, _CHUNK)])

    plsc.subcore_barrier()

    cbase = (cid * _NS + sid) * _NCH

    def i_start(j, s):
        pltpu.async_copy(ei_hbm.at[cbase + j], idx_v.at[s], isems[s])

    def i_wait(s):
        pltpu.make_async_copy(ei_hbm.at[cbase], idx_v.at[s], isems[s]).wait()

    def g_start(b, s):
        pltpu.async_copy(y_hbm.at[idx_v.at[s, 0]], rows_v.at[b], gsems[b])

    def g_wait(b, s):
        pltpu.make_async_copy(y_hbm.at[idx_v.at[s, 0]], rows_v.at[b],
                              gsems[b]).wait()

    def s_start(b, s):
        pltpu.async_copy(rows_v.at[b], acc_sh.at[idx_v.at[s, 1]], ssems[b],
                         add=True)

    def s_wait(b):
        pltpu.make_async_copy(rows_v.at[b], acc_sh.at[idx_v.at[0, 1]],
                              ssems[b]).wait()

    for s in range(4):
        i_start(s, s)
    for j in range(4):
        i_start(j + 4, j + 4)
        i_wait(j)
        g_start(j, j)
        if j >= 2:
            g_wait(j - 2, j - 2)
            s_start(j - 2, j - 2)
    for j in range(4, 8):
        s_wait(j % 4)
        i_start(j + 4, (j + 4) % 8)
        i_wait(j)
        g_start(j % 4, j)
        g_wait((j - 2) % 4, j - 2)
        s_start((j - 2) % 4, j - 2)

    @pl.loop(8, _NCH - 5, step=8)
    def _(j0):
        for k in range(8):
            j = j0 + k
            s_wait(k % 4)
            i_start(j + 4, (k + 4) % 8)
            i_wait(k)
            g_start(k % 4, k)
            g_wait((k - 2) % 4, (k - 2) % 8)
            s_start((k - 2) % 4, (k - 2) % 8)

    s_wait(0)
    i_start(_NCH - 1, 4)
    i_wait(0)
    g_start(0, 0)
    g_wait(2, 6)
    s_start(2, 6)
    for k in range(1, 4):
        s_wait(k)
        i_wait(k)
        g_start(k, k)
        g_wait((k + 2) % 4, (k + 6) % 8)
        s_start((k + 2) % 4, (k + 6) % 8)
    s_wait(0)
    i_wait(4)
    g_start(0, 4)
    g_wait(2, 2)
    s_start(2, 2)
    g_wait(3, 3)
    s_start(3, 3)
    g_wait(0, 4)
    s_start(0, 4)
    for b in (1, 2, 3, 0):
        s_wait(b)

    plsc.subcore_barrier()
    pltpu.sync_copy(
        acc_sh.at[pl.ds(sid * _RPS, _RPS)],
        acc_hbm.at[cid, pl.ds(sid * _RPS, _RPS)],
    )


@functools.lru_cache(maxsize=None)
def _sc_kernels():
    mesh = plsc.VectorSubcoreMesh(core_axis_name="c", subcore_axis_name="s",
                                  num_cores=_NC, num_subcores=_NS)
    deg_kernel = pl.kernel(
        _deg_body,
        out_type=jax.ShapeDtypeStruct((_NC, _NPAD), jnp.float32),
        mesh=mesh,
        scratch_types=[
            pltpu.VMEM((8, 2, _CHUNK), jnp.int32),
            pltpu.VMEM((_CHUNK,), jnp.float32),
            pltpu.VMEM_SHARED((_NPAD,), jnp.float32),
        ] + [pltpu.SemaphoreType.DMA] * 16,
    )
    gs_kernel = pl.kernel(
        _gs_body,
        out_type=jax.ShapeDtypeStruct((_NC, _NPAD, _H), jnp.float32),
        mesh=mesh,
        scratch_types=[
            pltpu.VMEM((8, 2, _CHUNK), jnp.int32),
            pltpu.VMEM((4, _CHUNK, _H), jnp.float32),
            pltpu.VMEM_SHARED((_NPAD, _H), jnp.float32),
        ] + [pltpu.SemaphoreType.DMA] * 16,
    )
    return deg_kernel, gs_kernel


_RB = 1000


def _y_body(dp_ref, x_ref, w_ref, y_ref):
    deg = dp_ref[0] + dp_ref[1] + 1.0
    xw = jnp.dot(x_ref[...], w_ref[...], preferred_element_type=jnp.float32)
    y_ref[...] = lax.rsqrt(deg) * xw


_y_call = pl.pallas_call(
    _y_body,
    grid=(_N // _RB,),
    in_specs=[
        pl.BlockSpec((_NC, _RB, 1), lambda i: (0, i, 0)),
        pl.BlockSpec((_RB, _D), lambda i: (i, 0)),
        pl.BlockSpec((_D, _H), lambda i: (0, 0)),
    ],
    out_specs=pl.BlockSpec((_RB, _H), lambda i: (i, 0)),
    out_shape=jax.ShapeDtypeStruct((_N, _H), jnp.float32),
)


def _epi_body(dp_ref, acc_ref, y_ref, b_ref, b1_ref, w2_ref, b2_ref,
              logits_ref, pool_ref, pool_acc):
    i = pl.program_id(0)

    @pl.when(i == 0)
    def _():
        pool_acc[...] = jnp.full((_G, _H), -jnp.inf, jnp.float32)

    deg = dp_ref[0] + dp_ref[1] + 1.0
    dis = lax.rsqrt(deg)
    h = dis * (acc_ref[0] + acc_ref[1] + y_ref[...]) + b1_ref[...][None, :]
    h = jnp.maximum(h, 0.0)
    bb = b_ref[...]
    bmin = jnp.min(bb)
    bmax = jnp.max(bb)
    for g in range(_G):
        @pl.when((g >= bmin) & (g <= bmax))
        def _():
            m = jnp.where(bb == g, h, -jnp.inf)
            pool_acc[g, :] = jnp.maximum(pool_acc[g, :], jnp.max(m, axis=0))

    @pl.when(i == pl.num_programs(0) - 1)
    def _():
        pool = pool_acc[...]
        pool_ref[...] = pool
        logits_ref[...] = (
            jnp.dot(pool, w2_ref[...], preferred_element_type=jnp.float32)
            + b2_ref[...][None, :]
        )


_epi_call = pl.pallas_call(
    _epi_body,
    grid=(_N // _RB,),
    in_specs=[
        pl.BlockSpec((_NC, _RB, 1), lambda i: (0, i, 0)),
        pl.BlockSpec((_NC, _RB, _H), lambda i: (0, i, 0)),
        pl.BlockSpec((_RB, _H), lambda i: (i, 0)),
        pl.BlockSpec((_RB, 1), lambda i: (i, 0)),
        pl.BlockSpec((_H,), lambda i: (0,)),
        pl.BlockSpec((_H, _OUT), lambda i: (0, 0)),
        pl.BlockSpec((_OUT,), lambda i: (0,)),
    ],
    out_specs=[
        pl.BlockSpec((_G, _OUT), lambda i: (0, 0)),
        pl.BlockSpec((_G, _H), lambda i: (0, 0)),
    ],
    out_shape=[
        jax.ShapeDtypeStruct((_G, _OUT), jnp.float32),
        jax.ShapeDtypeStruct((_G, _H), jnp.float32),
    ],
    scratch_shapes=[pltpu.VMEM((_G, _H), jnp.float32)],
)


def kernel(x, edge_index, batch, W1, b1, W2, b2):
    ei3 = edge_index.reshape(2, _E // _CHUNK, _CHUNK).transpose(1, 0, 2)
    deg_kernel, gs_kernel = _sc_kernels()
    deg_part = deg_kernel(ei3).reshape(_NC, _NPAD, 1)
    y = _y_call(deg_part, x, W1)
    acc = gs_kernel(ei3, y)
    logits, x_pool = _epi_call(deg_part, acc, y, batch.reshape(_N, 1),
                               b1, W2, b2)
    return (logits, x_pool)


# speedup vs baseline: 43.4576x; 1.0547x over previous
import functools

import jax
import jax.numpy as jnp
from jax import lax
from jax.experimental import pallas as pl
from jax.experimental.pallas import tpu as pltpu
from jax.experimental.pallas import tpu_sc as plsc

_N = 10000
_E = 320000
_D = 128
_H = 128
_OUT = 10
_G = 32

_NC = 2
_NS = 16
_LANES = 16
_NW = _NC * _NS

_E_TILE = _E // _NW
_CHUNK = 80
_NCH = _E_TILE // _CHUNK
_NPAD = 10240
_RPS = _NPAD // _NS


def _deg_body(ei_hbm, deg_hbm, idx_v, ones_v, deg_sh, *sems):
    cid = lax.axis_index("c")
    sid = lax.axis_index("s")
    isems = sems[0:8]
    ssems = sems[8:16]

    @pl.loop(0, _CHUNK, step=_LANES)
    def _(i):
        ones_v[pl.ds(i, _LANES)] = jnp.zeros((_LANES,), jnp.float32)

    @pl.loop(0, _RPS, step=_CHUNK)
    def _(r):
        pltpu.sync_copy(ones_v, deg_sh.at[pl.ds(sid * _RPS + r, _CHUNK)])

    @pl.loop(0, _CHUNK, step=_LANES)
    def _(i):
        ones_v[pl.ds(i, _LANES)] = jnp.full((_LANES,), 1.0, jnp.float32)

    plsc.subcore_barrier()

    cbase = (cid * _NS + sid) * _NCH

    def i_start(j, s):
        pltpu.async_copy(ei_hbm.at[cbase + j], idx_v.at[s], isems[s])

    def i_wait(s):
        pltpu.make_async_copy(ei_hbm.at[cbase], idx_v.at[s], isems[s]).wait()

    def s_start(s):
        pltpu.async_copy(ones_v, deg_sh.at[idx_v.at[s, 1]], ssems[s],
                         add=True)

    def s_wait(s):
        pltpu.make_async_copy(ones_v, deg_sh.at[idx_v.at[0, 1]],
                              ssems[s]).wait()

    for s in range(4):
        i_start(s, s)
    for j in range(4):
        i_start(j + 4, j + 4)
        i_wait(j)
        s_start(j)
    for j in range(4, 8):
        s_wait((j + 4) % 8)
        i_start(j + 4, (j + 4) % 8)
        i_wait(j)
        s_start(j)

    @pl.loop(8, _NCH - 5, step=8)
    def _(j0):
        for k in range(8):
            j = j0 + k
            s_wait((k + 4) % 8)
            i_start(j + 4, (k + 4) % 8)
            i_wait(k)
            s_start(k)

    s_wait(4)
    i_start(_NCH - 1, 4)
    i_wait(0)
    s_start(0)
    for k in range(1, 4):
        s_wait(k + 4)
        i_wait(k)
        s_start(k)
    s_wait(0)
    i_wait(4)
    s_start(4)
    for s in range(1, 5):
        s_wait(s)

    plsc.subcore_barrier()
    pltpu.sync_copy(
        deg_sh.at[pl.ds(sid * _RPS, _RPS)],
        deg_hbm.at[cid, pl.ds(sid * _RPS, _RPS)],
    )


def _gs_body(ei_hbm, y_hbm, acc_hbm, idx_v, rows_v, acc_sh, *sems):
    cid = lax.axis_index("c")
    sid = lax.axis_index("s")
    gsems = sems[0:4]
    ssems = sems[4:8]
    isems = sems[8:16]

    @pl.loop(0, _CHUNK)
    def _(i):
        @pl.loop(0, _H, step=_LANES)
        def _(j):
            rows_v[0, i, pl.ds(j, _LANES)] = jnp.zeros((_LANES,), jnp.float32)

    @pl.loop(0, _RPS, step=_CHUNK)
    def _(r):
        pltpu.sync_copy(rows_v.at[0], acc_sh.at[pl.ds(sid * _RPS + r, _CHUNK)])

    plsc.subcore_barrier()

    cbase = (cid * _NS + sid) * _NCH

    def i_start(j, s):
        pltpu.async_copy(ei_hbm.at[cbase + j], idx_v.at[s], isems[s])

    def i_wait(s):
        pltpu.make_async_copy(ei_hbm.at[cbase], idx_v.at[s], isems[s]).wait()

    def g_start(b, s):
        pltpu.async_copy(y_hbm.at[idx_v.at[s, 0]], rows_v.at[b], gsems[b])

    def g_wait(b, s):
        pltpu.make_async_copy(y_hbm.at[idx_v.at[s, 0]], rows_v.at[b],
                              gsems[b]).wait()

    def s_start(b, s):
        pltpu.async_copy(rows_v.at[b], acc_sh.at[idx_v.at[s, 1]], ssems[b],
                         add=True)

    def s_wait(b):
        pltpu.make_async_copy(rows_v.at[b], acc_sh.at[idx_v.at[0, 1]],
                              ssems[b]).wait()

    for s in range(4):
        i_start(s, s)
    for j in range(4):
        i_start(j + 4, j + 4)
        i_wait(j)
        g_start(j, j)
        if j >= 3:
            g_wait(j - 3, j - 3)
            s_start(j - 3, j - 3)
    for j in range(4, 8):
        s_wait(j % 4)
        i_start(j + 4, (j + 4) % 8)
        i_wait(j)
        g_start(j % 4, j)
        g_wait((j - 3) % 4, j - 3)
        s_start((j - 3) % 4, j - 3)

    @pl.loop(8, _NCH - 5, step=8)
    def _(j0):
        for k in range(8):
            j = j0 + k
            s_wait(k % 4)
            i_start(j + 4, (k + 4) % 8)
            i_wait(k)
            g_start(k % 4, k)
            g_wait((k + 1) % 4, (k + 5) % 8)
            s_start((k + 1) % 4, (k + 5) % 8)

    s_wait(0)
    i_start(_NCH - 1, 4)
    i_wait(0)
    g_start(0, 0)
    g_wait(1, 5)
    s_start(1, 5)
    for k in range(1, 4):
        s_wait(k)
        i_wait(k)
        g_start(k, k)
        g_wait((k + 1) % 4, (k + 5) % 8)
        s_start((k + 1) % 4, (k + 5) % 8)
    s_wait(0)
    i_wait(4)
    g_start(0, 4)
    g_wait(1, 1)
    s_start(1, 1)
    g_wait(2, 2)
    s_start(2, 2)
    g_wait(3, 3)
    s_start(3, 3)
    g_wait(0, 4)
    s_start(0, 4)
    for b in (1, 2, 3, 0):
        s_wait(b)

    plsc.subcore_barrier()
    pltpu.sync_copy(
        acc_sh.at[pl.ds(sid * _RPS, _RPS)],
        acc_hbm.at[cid, pl.ds(sid * _RPS, _RPS)],
    )


@functools.lru_cache(maxsize=None)
def _sc_kernels():
    mesh = plsc.VectorSubcoreMesh(core_axis_name="c", subcore_axis_name="s",
                                  num_cores=_NC, num_subcores=_NS)
    deg_kernel = pl.kernel(
        _deg_body,
        out_type=jax.ShapeDtypeStruct((_NC, _NPAD), jnp.float32),
        mesh=mesh,
        scratch_types=[
            pltpu.VMEM((8, 2, _CHUNK), jnp.int32),
            pltpu.VMEM((_CHUNK,), jnp.float32),
            pltpu.VMEM_SHARED((_NPAD,), jnp.float32),
        ] + [pltpu.SemaphoreType.DMA] * 16,
    )
    gs_kernel = pl.kernel(
        _gs_body,
        out_type=jax.ShapeDtypeStruct((_NC, _NPAD, _H), jnp.float32),
        mesh=mesh,
        scratch_types=[
            pltpu.VMEM((8, 2, _CHUNK), jnp.int32),
            pltpu.VMEM((4, _CHUNK, _H), jnp.float32),
            pltpu.VMEM_SHARED((_NPAD, _H), jnp.float32),
        ] + [pltpu.SemaphoreType.DMA] * 16,
    )
    return deg_kernel, gs_kernel


_RB = 1000


def _y_body(dp_ref, x_ref, w_ref, y_ref):
    deg = dp_ref[0] + dp_ref[1] + 1.0
    xw = jnp.dot(x_ref[...], w_ref[...], preferred_element_type=jnp.float32)
    y_ref[...] = lax.rsqrt(deg) * xw


_y_call = pl.pallas_call(
    _y_body,
    grid=(_N // _RB,),
    in_specs=[
        pl.BlockSpec((_NC, _RB, 1), lambda i: (0, i, 0)),
        pl.BlockSpec((_RB, _D), lambda i: (i, 0)),
        pl.BlockSpec((_D, _H), lambda i: (0, 0)),
    ],
    out_specs=pl.BlockSpec((_RB, _H), lambda i: (i, 0)),
    out_shape=jax.ShapeDtypeStruct((_N, _H), jnp.float32),
)


def _epi_body(dp_ref, acc_ref, y_ref, b_ref, b1_ref, w2_ref, b2_ref,
              logits_ref, pool_ref, pool_acc):
    i = pl.program_id(0)

    @pl.when(i == 0)
    def _():
        pool_acc[...] = jnp.full((_G, _H), -jnp.inf, jnp.float32)

    deg = dp_ref[0] + dp_ref[1] + 1.0
    dis = lax.rsqrt(deg)
    h = dis * (acc_ref[0] + acc_ref[1] + y_ref[...]) + b1_ref[...][None, :]
    h = jnp.maximum(h, 0.0)
    bb = b_ref[...]
    bmin = jnp.min(bb)
    bmax = jnp.max(bb)
    for g in range(_G):
        @pl.when((g >= bmin) & (g <= bmax))
        def _():
            m = jnp.where(bb == g, h, -jnp.inf)
            pool_acc[g, :] = jnp.maximum(pool_acc[g, :], jnp.max(m, axis=0))

    @pl.when(i == pl.num_programs(0) - 1)
    def _():
        pool = pool_acc[...]
        pool_ref[...] = pool
        logits_ref[...] = (
            jnp.dot(pool, w2_ref[...], preferred_element_type=jnp.float32)
            + b2_ref[...][None, :]
        )


_epi_call = pl.pallas_call(
    _epi_body,
    grid=(_N // _RB,),
    in_specs=[
        pl.BlockSpec((_NC, _RB, 1), lambda i: (0, i, 0)),
        pl.BlockSpec((_NC, _RB, _H), lambda i: (0, i, 0)),
        pl.BlockSpec((_RB, _H), lambda i: (i, 0)),
        pl.BlockSpec((_RB, 1), lambda i: (i, 0)),
        pl.BlockSpec((_H,), lambda i: (0,)),
        pl.BlockSpec((_H, _OUT), lambda i: (0, 0)),
        pl.BlockSpec((_OUT,), lambda i: (0,)),
    ],
    out_specs=[
        pl.BlockSpec((_G, _OUT), lambda i: (0, 0)),
        pl.BlockSpec((_G, _H), lambda i: (0, 0)),
    ],
    out_shape=[
        jax.ShapeDtypeStruct((_G, _OUT), jnp.float32),
        jax.ShapeDtypeStruct((_G, _H), jnp.float32),
    ],
    scratch_shapes=[pltpu.VMEM((_G, _H), jnp.float32)],
)


def kernel(x, edge_index, batch, W1, b1, W2, b2):
    ei3 = edge_index.reshape(2, _E // _CHUNK, _CHUNK).transpose(1, 0, 2)
    deg_kernel, gs_kernel = _sc_kernels()
    deg_part = deg_kernel(ei3).reshape(_NC, _NPAD, 1)
    y = _y_call(deg_part, x, W1)
    acc = gs_kernel(ei3, y)
    logits, x_pool = _epi_call(deg_part, acc, y, batch.reshape(_N, 1),
                               b1, W2, b2)
    return (logits, x_pool)


# speedup vs baseline: 43.8314x; 1.0086x over previous
import functools

import jax
import jax.numpy as jnp
from jax import lax
from jax.experimental import pallas as pl
from jax.experimental.pallas import tpu as pltpu
from jax.experimental.pallas import tpu_sc as plsc

_N = 10000
_E = 320000
_D = 128
_H = 128
_OUT = 10
_G = 32

_NC = 2
_NS = 16
_LANES = 16
_NW = _NC * _NS

_E_TILE = _E // _NW
_CHUNK = 80
_NCH = _E_TILE // _CHUNK
_NPAD = 10240
_RPS = _NPAD // _NS


def _deg_body(ei_hbm, deg_hbm, idx_v, ones_v, deg_sh, *sems):
    cid = lax.axis_index("c")
    sid = lax.axis_index("s")
    isems = sems[0:8]
    ssems = sems[8:16]

    @pl.loop(0, _CHUNK, step=_LANES)
    def _(i):
        ones_v[pl.ds(i, _LANES)] = jnp.zeros((_LANES,), jnp.float32)

    @pl.loop(0, _RPS, step=_CHUNK)
    def _(r):
        pltpu.sync_copy(ones_v, deg_sh.at[pl.ds(sid * _RPS + r, _CHUNK)])

    @pl.loop(0, _CHUNK, step=_LANES)
    def _(i):
        ones_v[pl.ds(i, _LANES)] = jnp.full((_LANES,), 1.0, jnp.float32)

    plsc.subcore_barrier()

    cbase = (cid * _NS + sid) * _NCH

    def i_start(j, s):
        pltpu.async_copy(ei_hbm.at[cbase + j], idx_v.at[s], isems[s])

    def i_wait(s):
        pltpu.make_async_copy(ei_hbm.at[cbase], idx_v.at[s], isems[s]).wait()

    def s_start(s):
        pltpu.async_copy(ones_v, deg_sh.at[idx_v.at[s, 1]], ssems[s],
                         add=True)

    def s_wait(s):
        pltpu.make_async_copy(ones_v, deg_sh.at[idx_v.at[0, 1]],
                              ssems[s]).wait()

    for s in range(4):
        i_start(s, s)
    for j in range(4):
        i_start(j + 4, j + 4)
        i_wait(j)
        s_start(j)
    for j in range(4, 8):
        s_wait((j + 4) % 8)
        i_start(j + 4, (j + 4) % 8)
        i_wait(j)
        s_start(j)

    @pl.loop(8, _NCH - 5, step=8)
    def _(j0):
        for k in range(8):
            j = j0 + k
            s_wait((k + 4) % 8)
            i_start(j + 4, (k + 4) % 8)
            i_wait(k)
            s_start(k)

    s_wait(4)
    i_start(_NCH - 1, 4)
    i_wait(0)
    s_start(0)
    for k in range(1, 4):
        s_wait(k + 4)
        i_wait(k)
        s_start(k)
    s_wait(0)
    i_wait(4)
    s_start(4)
    for s in range(1, 5):
        s_wait(s)

    plsc.subcore_barrier()
    pltpu.sync_copy(
        deg_sh.at[pl.ds(sid * _RPS, _RPS)],
        deg_hbm.at[cid, pl.ds(sid * _RPS, _RPS)],
    )


def _gs_body(ei_hbm, y_hbm, acc_hbm, idx_v, rows_v, acc_sh, *sems):
    cid = lax.axis_index("c")
    sid = lax.axis_index("s")
    gsems = sems[0:4]
    ssems = sems[4:8]
    isems = sems[8:16]

    @pl.loop(0, _CHUNK)
    def _(i):
        @pl.loop(0, _H, step=_LANES)
        def _(j):
            rows_v[0, i, pl.ds(j, _LANES)] = jnp.zeros((_LANES,), jnp.float32)

    @pl.loop(0, _RPS, step=_CHUNK)
    def _(r):
        pltpu.sync_copy(rows_v.at[0], acc_sh.at[pl.ds(sid * _RPS + r, _CHUNK)])

    plsc.subcore_barrier()

    cbase = (cid * _NS + sid) * _NCH

    def i_start(j, s):
        pltpu.async_copy(ei_hbm.at[cbase + j], idx_v.at[s], isems[s])

    def i_wait(s):
        pltpu.make_async_copy(ei_hbm.at[cbase], idx_v.at[s], isems[s]).wait()

    def g_start(b, s):
        pltpu.async_copy(y_hbm.at[idx_v.at[s, 0]], rows_v.at[b], gsems[b])

    def g_wait(b, s):
        pltpu.make_async_copy(y_hbm.at[idx_v.at[s, 0]], rows_v.at[b],
                              gsems[b]).wait()

    def s_start(b, s):
        pltpu.async_copy(rows_v.at[b], acc_sh.at[idx_v.at[s, 1]], ssems[b],
                         add=True)

    def s_wait(b):
        pltpu.make_async_copy(rows_v.at[b], acc_sh.at[idx_v.at[0, 1]],
                              ssems[b]).wait()

    for s in range(4):
        i_start(s, s)
    for j in range(4):
        i_start(j + 4, j + 4)
        i_wait(j)
        g_start(j, j)
        if j >= 3:
            g_wait(j - 3, j - 3)
            s_start(j - 3, j - 3)
    for j in range(4, 8):
        s_wait(j % 4)
        i_start(j + 4, (j + 4) % 8)
        i_wait(j)
        g_start(j % 4, j)
        g_wait((j - 3) % 4, j - 3)
        s_start((j - 3) % 4, j - 3)

    @pl.loop(8, _NCH - 5, step=8)
    def _(j0):
        for k in range(8):
            j = j0 + k
            s_wait(k % 4)
            i_start(j + 4, (k + 4) % 8)
            i_wait(k)
            g_start(k % 4, k)
            g_wait((k + 1) % 4, (k + 5) % 8)
            s_start((k + 1) % 4, (k + 5) % 8)

    s_wait(0)
    i_start(_NCH - 1, 4)
    i_wait(0)
    g_start(0, 0)
    g_wait(1, 5)
    s_start(1, 5)
    for k in range(1, 4):
        s_wait(k)
        i_wait(k)
        g_start(k, k)
        g_wait((k + 1) % 4, (k + 5) % 8)
        s_start((k + 1) % 4, (k + 5) % 8)
    s_wait(0)
    i_wait(4)
    g_start(0, 4)
    g_wait(1, 1)
    s_start(1, 1)
    g_wait(2, 2)
    s_start(2, 2)
    g_wait(3, 3)
    s_start(3, 3)
    g_wait(0, 4)
    s_start(0, 4)
    for b in (1, 2, 3, 0):
        s_wait(b)

    plsc.subcore_barrier()
    pltpu.sync_copy(
        acc_sh.at[pl.ds(sid * _RPS, _RPS)],
        acc_hbm.at[cid, pl.ds(sid * _RPS, _RPS)],
    )


@functools.lru_cache(maxsize=None)
def _sc_kernels():
    mesh = plsc.VectorSubcoreMesh(core_axis_name="c", subcore_axis_name="s",
                                  num_cores=_NC, num_subcores=_NS)
    deg_kernel = pl.kernel(
        _deg_body,
        out_type=jax.ShapeDtypeStruct((_NC, _NPAD), jnp.float32),
        mesh=mesh,
        scratch_types=[
            pltpu.VMEM((8, 2, _CHUNK), jnp.int32),
            pltpu.VMEM((_CHUNK,), jnp.float32),
            pltpu.VMEM_SHARED((_NPAD,), jnp.float32),
        ] + [pltpu.SemaphoreType.DMA] * 16,
    )
    gs_kernel = pl.kernel(
        _gs_body,
        out_type=jax.ShapeDtypeStruct((_NC, _NPAD, _H), jnp.float32),
        mesh=mesh,
        scratch_types=[
            pltpu.VMEM((8, 2, _CHUNK), jnp.int32),
            pltpu.VMEM((4, _CHUNK, _H), jnp.float32),
            pltpu.VMEM_SHARED((_NPAD, _H), jnp.float32),
        ] + [pltpu.SemaphoreType.DMA] * 16,
    )
    return deg_kernel, gs_kernel


_RB = 2000


def _y_body(dp_ref, x_ref, w_ref, y_ref):
    deg = dp_ref[0] + dp_ref[1] + 1.0
    xw = jnp.dot(x_ref[...], w_ref[...], preferred_element_type=jnp.float32)
    y_ref[...] = lax.rsqrt(deg) * xw


_y_call = pl.pallas_call(
    _y_body,
    grid=(_N // _RB,),
    in_specs=[
        pl.BlockSpec((_NC, _RB, 1), lambda i: (0, i, 0)),
        pl.BlockSpec((_RB, _D), lambda i: (i, 0)),
        pl.BlockSpec((_D, _H), lambda i: (0, 0)),
    ],
    out_specs=pl.BlockSpec((_RB, _H), lambda i: (i, 0)),
    out_shape=jax.ShapeDtypeStruct((_N, _H), jnp.float32),
)


def _epi_body(dp_ref, acc_ref, y_ref, b_ref, b1_ref, w2_ref, b2_ref,
              logits_ref, pool_ref, pool_acc):
    i = pl.program_id(0)

    @pl.when(i == 0)
    def _():
        pool_acc[...] = jnp.full((_G, _H), -jnp.inf, jnp.float32)

    deg = dp_ref[0] + dp_ref[1] + 1.0
    dis = lax.rsqrt(deg)
    h = dis * (acc_ref[0] + acc_ref[1] + y_ref[...]) + b1_ref[...][None, :]
    h = jnp.maximum(h, 0.0)
    bb = b_ref[...]
    bmin = jnp.min(bb)
    bmax = jnp.max(bb)
    for g in range(_G):
        @pl.when((g >= bmin) & (g <= bmax))
        def _():
            m = jnp.where(bb == g, h, -jnp.inf)
            pool_acc[g, :] = jnp.maximum(pool_acc[g, :], jnp.max(m, axis=0))

    @pl.when(i == pl.num_programs(0) - 1)
    def _():
        pool = pool_acc[...]
        pool_ref[...] = pool
        logits_ref[...] = (
            jnp.dot(pool, w2_ref[...], preferred_element_type=jnp.float32)
            + b2_ref[...][None, :]
        )


_epi_call = pl.pallas_call(
    _epi_body,
    grid=(_N // _RB,),
    in_specs=[
        pl.BlockSpec((_NC, _RB, 1), lambda i: (0, i, 0)),
        pl.BlockSpec((_NC, _RB, _H), lambda i: (0, i, 0)),
        pl.BlockSpec((_RB, _H), lambda i: (i, 0)),
        pl.BlockSpec((_RB, 1), lambda i: (i, 0)),
        pl.BlockSpec((_H,), lambda i: (0,)),
        pl.BlockSpec((_H, _OUT), lambda i: (0, 0)),
        pl.BlockSpec((_OUT,), lambda i: (0,)),
    ],
    out_specs=[
        pl.BlockSpec((_G, _OUT), lambda i: (0, 0)),
        pl.BlockSpec((_G, _H), lambda i: (0, 0)),
    ],
    out_shape=[
        jax.ShapeDtypeStruct((_G, _OUT), jnp.float32),
        jax.ShapeDtypeStruct((_G, _H), jnp.float32),
    ],
    scratch_shapes=[pltpu.VMEM((_G, _H), jnp.float32)],
)


def kernel(x, edge_index, batch, W1, b1, W2, b2):
    ei3 = edge_index.reshape(2, _E // _CHUNK, _CHUNK).transpose(1, 0, 2)
    deg_kernel, gs_kernel = _sc_kernels()
    deg_part = deg_kernel(ei3).reshape(_NC, _NPAD, 1)
    y = _y_call(deg_part, x, W1)
    acc = gs_kernel(ei3, y)
    logits, x_pool = _epi_call(deg_part, acc, y, batch.reshape(_N, 1),
                               b1, W2, b2)
    return (logits, x_pool)


# speedup vs baseline: 44.0420x; 1.0048x over previous
import functools

import jax
import jax.numpy as jnp
from jax import lax
from jax.experimental import pallas as pl
from jax.experimental.pallas import tpu as pltpu
from jax.experimental.pallas import tpu_sc as plsc

_N = 10000
_E = 320000
_D = 128
_H = 128
_OUT = 10
_G = 32

_NC = 2
_NS = 16
_LANES = 16
_NW = _NC * _NS

_E_TILE = _E // _NW
_CHUNK = 80
_NCH = _E_TILE // _CHUNK
_NPAD = 10240
_RPS = _NPAD // _NS


def _deg_body(ei_hbm, deg_hbm, idx_v, ones_v, deg_sh, *sems):
    cid = lax.axis_index("c")
    sid = lax.axis_index("s")
    isems = sems[0:8]
    ssems = sems[8:16]
    cbase = (cid * _NS + sid) * _NCH

    def i_start(j, s):
        pltpu.async_copy(ei_hbm.at[cbase + j], idx_v.at[s], isems[s])

    for s in range(4):
        i_start(s, s)

    @pl.loop(0, _CHUNK, step=_LANES)
    def _(i):
        ones_v[pl.ds(i, _LANES)] = jnp.zeros((_LANES,), jnp.float32)

    @pl.loop(0, _RPS, step=_CHUNK)
    def _(r):
        pltpu.sync_copy(ones_v, deg_sh.at[pl.ds(sid * _RPS + r, _CHUNK)])

    @pl.loop(0, _CHUNK, step=_LANES)
    def _(i):
        ones_v[pl.ds(i, _LANES)] = jnp.full((_LANES,), 1.0, jnp.float32)

    plsc.subcore_barrier()

    def i_wait(s):
        pltpu.make_async_copy(ei_hbm.at[cbase], idx_v.at[s], isems[s]).wait()

    def s_start(s):
        pltpu.async_copy(ones_v, deg_sh.at[idx_v.at[s, 1]], ssems[s],
                         add=True)

    def s_wait(s):
        pltpu.make_async_copy(ones_v, deg_sh.at[idx_v.at[0, 1]],
                              ssems[s]).wait()

    for j in range(4):
        i_start(j + 4, j + 4)
        i_wait(j)
        s_start(j)
    for j in range(4, 8):
        s_wait((j + 4) % 8)
        i_start(j + 4, (j + 4) % 8)
        i_wait(j)
        s_start(j)

    @pl.loop(8, _NCH - 5, step=8)
    def _(j0):
        for k in range(8):
            j = j0 + k
            s_wait((k + 4) % 8)
            i_start(j + 4, (k + 4) % 8)
            i_wait(k)
            s_start(k)

    s_wait(4)
    i_start(_NCH - 1, 4)
    i_wait(0)
    s_start(0)
    for k in range(1, 4):
        s_wait(k + 4)
        i_wait(k)
        s_start(k)
    s_wait(0)
    i_wait(4)
    s_start(4)
    for s in range(1, 5):
        s_wait(s)

    plsc.subcore_barrier()
    pltpu.sync_copy(
        deg_sh.at[pl.ds(sid * _RPS, _RPS)],
        deg_hbm.at[cid, pl.ds(sid * _RPS, _RPS)],
    )


def _gs_body(ei_hbm, y_hbm, acc_hbm, idx_v, rows_v, acc_sh, *sems):
    cid = lax.axis_index("c")
    sid = lax.axis_index("s")
    gsems = sems[0:4]
    ssems = sems[4:8]
    isems = sems[8:16]
    cbase = (cid * _NS + sid) * _NCH

    def i_start(j, s):
        pltpu.async_copy(ei_hbm.at[cbase + j], idx_v.at[s], isems[s])

    for s in range(4):
        i_start(s, s)

    @pl.loop(0, _CHUNK)
    def _(i):
        @pl.loop(0, _H, step=_LANES)
        def _(j):
            rows_v[0, i, pl.ds(j, _LANES)] = jnp.zeros((_LANES,), jnp.float32)

    @pl.loop(0, _RPS, step=_CHUNK)
    def _(r):
        pltpu.sync_copy(rows_v.at[0], acc_sh.at[pl.ds(sid * _RPS + r, _CHUNK)])

    plsc.subcore_barrier()

    def i_wait(s):
        pltpu.make_async_copy(ei_hbm.at[cbase], idx_v.at[s], isems[s]).wait()

    def g_start(b, s):
        pltpu.async_copy(y_hbm.at[idx_v.at[s, 0]], rows_v.at[b], gsems[b])

    def g_wait(b, s):
        pltpu.make_async_copy(y_hbm.at[idx_v.at[s, 0]], rows_v.at[b],
                              gsems[b]).wait()

    def s_start(b, s):
        pltpu.async_copy(rows_v.at[b], acc_sh.at[idx_v.at[s, 1]], ssems[b],
                         add=True)

    def s_wait(b):
        pltpu.make_async_copy(rows_v.at[b], acc_sh.at[idx_v.at[0, 1]],
                              ssems[b]).wait()

    for j in range(4):
        i_start(j + 4, j + 4)
        i_wait(j)
        g_start(j, j)
        if j >= 3:
            g_wait(j - 3, j - 3)
            s_start(j - 3, j - 3)
    for j in range(4, 8):
        s_wait(j % 4)
        i_start(j + 4, (j + 4) % 8)
        i_wait(j)
        g_start(j % 4, j)
        g_wait((j - 3) % 4, j - 3)
        s_start((j - 3) % 4, j - 3)

    @pl.loop(8, _NCH - 5, step=8)
    def _(j0):
        for k in range(8):
            j = j0 + k
            s_wait(k % 4)
            i_start(j + 4, (k + 4) % 8)
            i_wait(k)
            g_start(k % 4, k)
            g_wait((k + 1) % 4, (k + 5) % 8)
            s_start((k + 1) % 4, (k + 5) % 8)

    s_wait(0)
    i_start(_NCH - 1, 4)
    i_wait(0)
    g_start(0, 0)
    g_wait(1, 5)
    s_start(1, 5)
    for k in range(1, 4):
        s_wait(k)
        i_wait(k)
        g_start(k, k)
        g_wait((k + 1) % 4, (k + 5) % 8)
        s_start((k + 1) % 4, (k + 5) % 8)
    s_wait(0)
    i_wait(4)
    g_start(0, 4)
    g_wait(1, 1)
    s_start(1, 1)
    g_wait(2, 2)
    s_start(2, 2)
    g_wait(3, 3)
    s_start(3, 3)
    g_wait(0, 4)
    s_start(0, 4)
    for b in (1, 2, 3, 0):
        s_wait(b)

    plsc.subcore_barrier()
    pltpu.sync_copy(
        acc_sh.at[pl.ds(sid * _RPS, _RPS)],
        acc_hbm.at[cid, pl.ds(sid * _RPS, _RPS)],
    )


@functools.lru_cache(maxsize=None)
def _sc_kernels():
    mesh = plsc.VectorSubcoreMesh(core_axis_name="c", subcore_axis_name="s",
                                  num_cores=_NC, num_subcores=_NS)
    deg_kernel = pl.kernel(
        _deg_body,
        out_type=jax.ShapeDtypeStruct((_NC, _NPAD), jnp.float32),
        mesh=mesh,
        scratch_types=[
            pltpu.VMEM((8, 2, _CHUNK), jnp.int32),
            pltpu.VMEM((_CHUNK,), jnp.float32),
            pltpu.VMEM_SHARED((_NPAD,), jnp.float32),
        ] + [pltpu.SemaphoreType.DMA] * 16,
    )
    gs_kernel = pl.kernel(
        _gs_body,
        out_type=jax.ShapeDtypeStruct((_NC, _NPAD, _H), jnp.float32),
        mesh=mesh,
        scratch_types=[
            pltpu.VMEM((8, 2, _CHUNK), jnp.int32),
            pltpu.VMEM((4, _CHUNK, _H), jnp.float32),
            pltpu.VMEM_SHARED((_NPAD, _H), jnp.float32),
        ] + [pltpu.SemaphoreType.DMA] * 16,
    )
    return deg_kernel, gs_kernel


_RB = 2000


def _y_body(dp_ref, x_ref, w_ref, y_ref):
    deg = dp_ref[0] + dp_ref[1] + 1.0
    xw = jnp.dot(x_ref[...], w_ref[...], preferred_element_type=jnp.float32)
    y_ref[...] = lax.rsqrt(deg) * xw


_y_call = pl.pallas_call(
    _y_body,
    grid=(_N // _RB,),
    in_specs=[
        pl.BlockSpec((_NC, _RB, 1), lambda i: (0, i, 0)),
        pl.BlockSpec((_RB, _D), lambda i: (i, 0)),
        pl.BlockSpec((_D, _H), lambda i: (0, 0)),
    ],
    out_specs=pl.BlockSpec((_RB, _H), lambda i: (i, 0)),
    out_shape=jax.ShapeDtypeStruct((_N, _H), jnp.float32),
)


def _epi_body(dp_ref, acc_ref, y_ref, b_ref, b1_ref, w2_ref, b2_ref,
              logits_ref, pool_ref, pool_acc):
    i = pl.program_id(0)

    @pl.when(i == 0)
    def _():
        pool_acc[...] = jnp.full((_G, _H), -jnp.inf, jnp.float32)

    deg = dp_ref[0] + dp_ref[1] + 1.0
    dis = lax.rsqrt(deg)
    h = dis * (acc_ref[0] + acc_ref[1] + y_ref[...]) + b1_ref[...][None, :]
    h = jnp.maximum(h, 0.0)
    bb = b_ref[...]
    bmin = jnp.min(bb)
    bmax = jnp.max(bb)
    for g in range(_G):
        @pl.when((g >= bmin) & (g <= bmax))
        def _():
            m = jnp.where(bb == g, h, -jnp.inf)
            pool_acc[g, :] = jnp.maximum(pool_acc[g, :], jnp.max(m, axis=0))

    @pl.when(i == pl.num_programs(0) - 1)
    def _():
        pool = pool_acc[...]
        pool_ref[...] = pool
        logits_ref[...] = (
            jnp.dot(pool, w2_ref[...], preferred_element_type=jnp.float32)
            + b2_ref[...][None, :]
        )


_epi_call = pl.pallas_call(
    _epi_body,
    grid=(_N // _RB,),
    in_specs=[
        pl.BlockSpec((_NC, _RB, 1), lambda i: (0, i, 0)),
        pl.BlockSpec((_NC, _RB, _H), lambda i: (0, i, 0)),
        pl.BlockSpec((_RB, _H), lambda i: (i, 0)),
        pl.BlockSpec((_RB, 1), lambda i: (i, 0)),
        pl.BlockSpec((_H,), lambda i: (0,)),
        pl.BlockSpec((_H, _OUT), lambda i: (0, 0)),
        pl.BlockSpec((_OUT,), lambda i: (0,)),
    ],
    out_specs=[
        pl.BlockSpec((_G, _OUT), lambda i: (0, 0)),
        pl.BlockSpec((_G, _H), lambda i: (0, 0)),
    ],
    out_shape=[
        jax.ShapeDtypeStruct((_G, _OUT), jnp.float32),
        jax.ShapeDtypeStruct((_G, _H), jnp.float32),
    ],
    scratch_shapes=[pltpu.VMEM((_G, _H), jnp.float32)],
)


def kernel(x, edge_index, batch, W1, b1, W2, b2):
    ei3 = edge_index.reshape(2, _E // _CHUNK, _CHUNK).transpose(1, 0, 2)
    deg_kernel, gs_kernel = _sc_kernels()
    deg_part = deg_kernel(ei3).reshape(_NC, _NPAD, 1)
    y = _y_call(deg_part, x, W1)
    acc = gs_kernel(ei3, y)
    logits, x_pool = _epi_call(deg_part, acc, y, batch.reshape(_N, 1),
                               b1, W2, b2)
    return (logits, x_pool)


# speedup vs baseline: 44.1540x; 1.0025x over previous
import functools

import jax
import jax.numpy as jnp
from jax import lax
from jax.experimental import pallas as pl
from jax.experimental.pallas import tpu as pltpu
from jax.experimental.pallas import tpu_sc as plsc

_N = 10000
_E = 320000
_D = 128
_H = 128
_OUT = 10
_G = 32

_NC = 2
_NS = 16
_LANES = 16
_NW = _NC * _NS

_E_TILE = _E // _NW
_CHUNK = 80
_NCH = _E_TILE // _CHUNK
_NPAD = 10240
_RPS = _NPAD // _NS


def _deg_body(ei_hbm, deg_hbm, idx_v, ones_v, deg_sh, *sems):
    cid = lax.axis_index("c")
    sid = lax.axis_index("s")
    isems = sems[0:8]
    ssems = sems[8:16]
    cbase = (cid * _NS + sid) * _NCH

    def i_start(j, s):
        pltpu.async_copy(ei_hbm.at[cbase + j], idx_v.at[s], isems[s])

    for s in range(4):
        i_start(s, s)

    @pl.loop(0, _CHUNK, step=_LANES)
    def _(i):
        ones_v[pl.ds(i, _LANES)] = jnp.zeros((_LANES,), jnp.float32)

    @pl.loop(0, _RPS, step=_CHUNK)
    def _(r):
        pltpu.sync_copy(ones_v, deg_sh.at[pl.ds(sid * _RPS + r, _CHUNK)])

    @pl.loop(0, _CHUNK, step=_LANES)
    def _(i):
        ones_v[pl.ds(i, _LANES)] = jnp.full((_LANES,), 1.0, jnp.float32)

    plsc.subcore_barrier()

    def i_wait(s):
        pltpu.make_async_copy(ei_hbm.at[cbase], idx_v.at[s], isems[s]).wait()

    def s_start(s):
        pltpu.async_copy(ones_v, deg_sh.at[idx_v.at[s, 1]], ssems[s],
                         add=True)

    def s_wait(s):
        pltpu.make_async_copy(ones_v, deg_sh.at[idx_v.at[0, 1]],
                              ssems[s]).wait()

    for j in range(4):
        i_start(j + 4, j + 4)
        i_wait(j)
        s_start(j)
    for j in range(4, 8):
        s_wait((j + 4) % 8)
        i_start(j + 4, (j + 4) % 8)
        i_wait(j)
        s_start(j)

    @pl.loop(8, _NCH - 5, step=8)
    def _(j0):
        for k in range(8):
            j = j0 + k
            s_wait((k + 4) % 8)
            i_start(j + 4, (k + 4) % 8)
            i_wait(k)
            s_start(k)

    s_wait(4)
    i_start(_NCH - 1, 4)
    i_wait(0)
    s_start(0)
    for k in range(1, 4):
        s_wait(k + 4)
        i_wait(k)
        s_start(k)
    s_wait(0)
    i_wait(4)
    s_start(4)
    for s in range(1, 5):
        s_wait(s)

    plsc.subcore_barrier()
    pltpu.sync_copy(
        deg_sh.at[pl.ds(sid * _RPS, _RPS)],
        deg_hbm.at[cid, pl.ds(sid * _RPS, _RPS)],
    )


def _gs_body(ei_hbm, y_hbm, acc_hbm, idx_v, rows_v, acc_sh, *sems):
    cid = lax.axis_index("c")
    sid = lax.axis_index("s")
    gsems = sems[0:4]
    ssems = sems[4:8]
    isems = sems[8:16]
    cbase = (cid * _NS + sid) * _NCH

    def i_start(j, s):
        pltpu.async_copy(ei_hbm.at[cbase + j], idx_v.at[s], isems[s])

    for s in range(4):
        i_start(s, s)

    @pl.loop(0, _CHUNK)
    def _(i):
        @pl.loop(0, _H, step=_LANES)
        def _(j):
            rows_v[0, i, pl.ds(j, _LANES)] = jnp.zeros((_LANES,), jnp.float32)

    @pl.loop(0, _RPS, step=_CHUNK)
    def _(r):
        pltpu.async_copy(rows_v.at[0],
                         acc_sh.at[pl.ds(sid * _RPS + r, _CHUNK)], sems[4])

    @pl.loop(0, _RPS, step=_CHUNK)
    def _(r):
        pltpu.make_async_copy(rows_v.at[0],
                              acc_sh.at[pl.ds(sid * _RPS, _CHUNK)],
                              sems[4]).wait()

    plsc.subcore_barrier()

    def i_wait(s):
        pltpu.make_async_copy(ei_hbm.at[cbase], idx_v.at[s], isems[s]).wait()

    def g_start(b, s):
        pltpu.async_copy(y_hbm.at[idx_v.at[s, 0]], rows_v.at[b], gsems[b])

    def g_wait(b, s):
        pltpu.make_async_copy(y_hbm.at[idx_v.at[s, 0]], rows_v.at[b],
                              gsems[b]).wait()

    def s_start(b, s):
        pltpu.async_copy(rows_v.at[b], acc_sh.at[idx_v.at[s, 1]], ssems[b],
                         add=True)

    def s_wait(b):
        pltpu.make_async_copy(rows_v.at[b], acc_sh.at[idx_v.at[0, 1]],
                              ssems[b]).wait()

    for j in range(4):
        i_start(j + 4, j + 4)
        i_wait(j)
        g_start(j, j)
        if j >= 3:
            g_wait(j - 3, j - 3)
            s_start(j - 3, j - 3)
    for j in range(4, 8):
        s_wait(j % 4)
        i_start(j + 4, (j + 4) % 8)
        i_wait(j)
        g_start(j % 4, j)
        g_wait((j - 3) % 4, j - 3)
        s_start((j - 3) % 4, j - 3)

    @pl.loop(8, _NCH - 5, step=8)
    def _(j0):
        for k in range(8):
            j = j0 + k
            s_wait(k % 4)
            i_start(j + 4, (k + 4) % 8)
            i_wait(k)
            g_start(k % 4, k)
            g_wait((k + 1) % 4, (k + 5) % 8)
            s_start((k + 1) % 4, (k + 5) % 8)

    s_wait(0)
    i_start(_NCH - 1, 4)
    i_wait(0)
    g_start(0, 0)
    g_wait(1, 5)
    s_start(1, 5)
    for k in range(1, 4):
        s_wait(k)
        i_wait(k)
        g_start(k, k)
        g_wait((k + 1) % 4, (k + 5) % 8)
        s_start((k + 1) % 4, (k + 5) % 8)
    s_wait(0)
    i_wait(4)
    g_start(0, 4)
    g_wait(1, 1)
    s_start(1, 1)
    g_wait(2, 2)
    s_start(2, 2)
    g_wait(3, 3)
    s_start(3, 3)
    g_wait(0, 4)
    s_start(0, 4)
    for b in (1, 2, 3, 0):
        s_wait(b)

    plsc.subcore_barrier()
    pltpu.sync_copy(
        acc_sh.at[pl.ds(sid * _RPS, _RPS)],
        acc_hbm.at[cid, pl.ds(sid * _RPS, _RPS)],
    )


@functools.lru_cache(maxsize=None)
def _sc_kernels():
    mesh = plsc.VectorSubcoreMesh(core_axis_name="c", subcore_axis_name="s",
                                  num_cores=_NC, num_subcores=_NS)
    deg_kernel = pl.kernel(
        _deg_body,
        out_type=jax.ShapeDtypeStruct((_NC, _NPAD), jnp.float32),
        mesh=mesh,
        scratch_types=[
            pltpu.VMEM((8, 2, _CHUNK), jnp.int32),
            pltpu.VMEM((_CHUNK,), jnp.float32),
            pltpu.VMEM_SHARED((_NPAD,), jnp.float32),
        ] + [pltpu.SemaphoreType.DMA] * 16,
    )
    gs_kernel = pl.kernel(
        _gs_body,
        out_type=jax.ShapeDtypeStruct((_NC, _NPAD, _H), jnp.float32),
        mesh=mesh,
        scratch_types=[
            pltpu.VMEM((8, 2, _CHUNK), jnp.int32),
            pltpu.VMEM((4, _CHUNK, _H), jnp.float32),
            pltpu.VMEM_SHARED((_NPAD, _H), jnp.float32),
        ] + [pltpu.SemaphoreType.DMA] * 16,
    )
    return deg_kernel, gs_kernel


_RB = 2000


def _y_body(dp_ref, x_ref, w_ref, y_ref):
    deg = dp_ref[0] + dp_ref[1] + 1.0
    xw = jnp.dot(x_ref[...], w_ref[...], preferred_element_type=jnp.float32)
    y_ref[...] = lax.rsqrt(deg) * xw


_y_call = pl.pallas_call(
    _y_body,
    grid=(_N // _RB,),
    in_specs=[
        pl.BlockSpec((_NC, _RB, 1), lambda i: (0, i, 0)),
        pl.BlockSpec((_RB, _D), lambda i: (i, 0)),
        pl.BlockSpec((_D, _H), lambda i: (0, 0)),
    ],
    out_specs=pl.BlockSpec((_RB, _H), lambda i: (i, 0)),
    out_shape=jax.ShapeDtypeStruct((_N, _H), jnp.float32),
)


def _epi_body(dp_ref, acc_ref, y_ref, b_ref, b1_ref, w2_ref, b2_ref,
              logits_ref, pool_ref, pool_acc):
    i = pl.program_id(0)

    @pl.when(i == 0)
    def _():
        pool_acc[...] = jnp.full((_G, _H), -jnp.inf, jnp.float32)

    deg = dp_ref[0] + dp_ref[1] + 1.0
    dis = lax.rsqrt(deg)
    h = dis * (acc_ref[0] + acc_ref[1] + y_ref[...]) + b1_ref[...][None, :]
    h = jnp.maximum(h, 0.0)
    bb = b_ref[...]
    bmin = jnp.min(bb)
    bmax = jnp.max(bb)
    for g in range(_G):
        @pl.when((g >= bmin) & (g <= bmax))
        def _():
            m = jnp.where(bb == g, h, -jnp.inf)
            pool_acc[g, :] = jnp.maximum(pool_acc[g, :], jnp.max(m, axis=0))

    @pl.when(i == pl.num_programs(0) - 1)
    def _():
        pool = pool_acc[...]
        pool_ref[...] = pool
        logits_ref[...] = (
            jnp.dot(pool, w2_ref[...], preferred_element_type=jnp.float32)
            + b2_ref[...][None, :]
        )


_epi_call = pl.pallas_call(
    _epi_body,
    grid=(_N // _RB,),
    in_specs=[
        pl.BlockSpec((_NC, _RB, 1), lambda i: (0, i, 0)),
        pl.BlockSpec((_NC, _RB, _H), lambda i: (0, i, 0)),
        pl.BlockSpec((_RB, _H), lambda i: (i, 0)),
        pl.BlockSpec((_RB, 1), lambda i: (i, 0)),
        pl.BlockSpec((_H,), lambda i: (0,)),
        pl.BlockSpec((_H, _OUT), lambda i: (0, 0)),
        pl.BlockSpec((_OUT,), lambda i: (0,)),
    ],
    out_specs=[
        pl.BlockSpec((_G, _OUT), lambda i: (0, 0)),
        pl.BlockSpec((_G, _H), lambda i: (0, 0)),
    ],
    out_shape=[
        jax.ShapeDtypeStruct((_G, _OUT), jnp.float32),
        jax.ShapeDtypeStruct((_G, _H), jnp.float32),
    ],
    scratch_shapes=[pltpu.VMEM((_G, _H), jnp.float32)],
)


def kernel(x, edge_index, batch, W1, b1, W2, b2):
    ei3 = edge_index.reshape(2, _E // _CHUNK, _CHUNK).transpose(1, 0, 2)
    deg_kernel, gs_kernel = _sc_kernels()
    deg_part = deg_kernel(ei3).reshape(_NC, _NPAD, 1)
    y = _y_call(deg_part, x, W1)
    acc = gs_kernel(ei3, y)
    logits, x_pool = _epi_call(deg_part, acc, y, batch.reshape(_N, 1),
                               b1, W2, b2)
    return (logits, x_pool)
